# baseline (speedup 1.0000x reference)
import jax
import jax.numpy as jnp
from jax.experimental import pallas as pl


def _gcn_conv(x, ei, W, b):
    n = x.shape[0]
    loop = jnp.arange(n, dtype=ei.dtype)
    src = jnp.concatenate([ei[0], loop])
    dst = jnp.concatenate([ei[1], loop])
    deg = jnp.zeros((n,), dtype=x.dtype).at[dst].add(1.0)
    dinv = jax.lax.rsqrt(jnp.maximum(deg, 1e-12))
    norm = dinv[src] * dinv[dst]
    h = x @ W
    msg = h[src] * norm[:, None]
    out = jnp.zeros((n, W.shape[1]), dtype=x.dtype).at[dst].add(msg)
    return out + b


def _proj_body(h_ref, w_ref, b_ref, o_ref):
    o_ref[...] = jnp.sum(h_ref[...] * w_ref[...].T, axis=1, keepdims=True) + b_ref[...]


def kernel(x, edge1, pos, idx, ei2, emb, W1, b1, W2a, b2a, W2b, b2b, Wp, bp):
    h = emb[x]
    h = jax.nn.relu(_gcn_conv(h, edge1, W1, b1))
    h = h[pos[:, 0]] * h[pos[:, 1]]
    ei2_r = ei2[::-1]
    h = jax.nn.relu(_gcn_conv(h, ei2, W2a, b2a)) + jax.nn.relu(_gcn_conv(h, ei2_r, W2b, b2b))
    h = h[idx]
    h = h[0::2] * h[1::2]
    return pl.pallas_call(
        _proj_body,
        out_shape=jax.ShapeDtypeStruct((h.shape[0], 1), h.dtype),
    )(h, Wp, bp)


# speedup vs baseline: 35.8424x; 35.8424x over previous
import jax
import jax.numpy as jnp
from jax import lax
from jax.experimental import pallas as pl
from jax.experimental.pallas import tpu as pltpu
from jax.experimental.pallas import tpu_sc as plsc

F32 = jnp.float32
I32 = jnp.int32
HI = lax.Precision.HIGHEST

_N = 10000
_N2 = 10240
_NR = _N2 // 128
_E1PAD = 655360
_E1R = _E1PAD // 128
_P = 100000
_P2 = 102400
_PR = _P2 // 128
_E2PAD = 1605632
_E2R = _E2PAD // 128
_F = 50000
_F2 = 53248
_FR = _F2 // 128
_H = 16

_NC, _NS = 2, 16
_MESH = plsc.VectorSubcoreMesh(
    core_axis_name="c", subcore_axis_name="s", num_cores=_NC, num_subcores=_NS)
_SC_PARAMS = pltpu.CompilerParams(use_tc_tiling_on_sc=False)


def _rs(d):
    return lax.rsqrt(jnp.maximum(d, 1e-12))


def _count_gather_body(d1r, s2r, d2r, xr, embw,
                       deg1p, degap, degbp, g1,
                       deg1_sh, dega_sh, degb_sh,
                       ones_v, zero_v, idx_v, gidx_v, grow_v, sem):
    cid = lax.axis_index("c")
    sid = lax.axis_index("s")

    for i in range(8):
        ones_v[pl.ds(i * 16, 16)] = jnp.ones((16,), F32)

    def _z(i, c):
        zero_v[pl.ds(i * 16, 16)] = jnp.zeros((16,), F32)
        return c
    lax.fori_loop(0, 400, _z, 0)

    pltpu.sync_copy(zero_v.at[pl.ds(0, 640)], deg1_sh.at[pl.ds(sid * 640, 640)])
    pltpu.sync_copy(zero_v, dega_sh.at[pl.ds(sid * 6400, 6400)])
    pltpu.sync_copy(zero_v, degb_sh.at[pl.ds(sid * 6400, 6400)])
    plsc.subcore_barrier()

    def _count(idx_hbm, acc_sh, nrows_tile):
        base = cid * (_NS * nrows_tile) + sid * nrows_tile

        def body(g, c):
            pltpu.sync_copy(idx_hbm.at[pl.ds(base + g * 8, 8)], idx_v)
            descs = [pltpu.async_copy(ones_v, acc_sh.at[idx_v.at[j]], sem,
                                      add=True) for j in range(8)]
            for d in descs:
                d.wait()
            return c
        lax.fori_loop(0, nrows_tile // 8, body, 0)

    _count(d1r, deg1_sh, 160)
    _count(d2r, dega_sh, 392)
    _count(s2r, degb_sh, 392)

    wid = sid * _NC + cid

    @pl.when(wid < 20)
    def _():
        base = wid * 4
        pltpu.sync_copy(xr.at[pl.ds(base, 4)], gidx_v)
        descs = [pltpu.async_copy(embw.at[gidx_v.at[j]], grow_v.at[j], sem)
                 for j in range(4)]
        for d in descs:
            d.wait()
        descs = [pltpu.async_copy(grow_v.at[j], g1.at[pl.ds((base + j) * 128, 128)],
                                  sem) for j in range(4)]
        for d in descs:
            d.wait()

    plsc.subcore_barrier()
    pltpu.sync_copy(deg1_sh.at[pl.ds(sid * 640, 640)],
                    deg1p.at[cid].at[pl.ds(sid * 640, 640)])
    pltpu.sync_copy(dega_sh.at[pl.ds(sid * 6400, 6400)],
                    degap.at[cid].at[pl.ds(sid * 6400, 6400)])
    pltpu.sync_copy(degb_sh.at[pl.ds(sid * 6400, 6400)],
                    degbp.at[cid].at[pl.ds(sid * 6400, 6400)])


def _conv1_body(s1r, d1r, hw1p, accp,
                acc_sh, si_v, di_v, rows_v, semg, sems):
    cid = lax.axis_index("c")
    sid = lax.axis_index("s")
    pltpu.sync_copy(hw1p.at[pl.ds(sid * 640, 640)], acc_sh.at[pl.ds(sid * 640, 640)])
    plsc.subcore_barrier()
    base = cid * (_NS * 160) + sid * 160

    def body(g, c):
        pltpu.sync_copy(s1r.at[pl.ds(base + g * 8, 8)], si_v)
        pltpu.sync_copy(d1r.at[pl.ds(base + g * 8, 8)], di_v)
        ds1 = [pltpu.async_copy(hw1p.at[si_v.at[j]], rows_v.at[j], semg)
               for j in range(8)]
        for d in ds1:
            d.wait()
        ds2 = [pltpu.async_copy(rows_v.at[j], acc_sh.at[di_v.at[j]], sems,
                                add=True) for j in range(8)]
        for d in ds2:
            d.wait()
        return c
    lax.fori_loop(0, 20, body, 0)
    plsc.subcore_barrier()
    pltpu.sync_copy(acc_sh.at[pl.ds(sid * 640, 640)],
                    accp.at[cid].at[pl.ds(sid * 640, 640)])


def _conv2_body(gir, sir, tbl2, acc2,
                acc_sh, gi_v, si_v, rows_v, semg, sems):
    cid = lax.axis_index("c")
    sid = lax.axis_index("s")
    pltpu.sync_copy(tbl2.at[cid].at[pl.ds(sid * 6400, 6400)],
                    acc_sh.at[pl.ds(sid * 6400, 6400)])
    plsc.subcore_barrier()
    base = sid * 784

    def body(g, c):
        pltpu.sync_copy(gir.at[pl.ds(base + g * 8, 8)], gi_v)
        pltpu.sync_copy(sir.at[pl.ds(base + g * 8, 8)], si_v)
        ds1 = [pltpu.async_copy(tbl2.at[cid].at[gi_v.at[j]], rows_v.at[j], semg)
               for j in range(8)]
        for d in ds1:
            d.wait()
        ds2 = [pltpu.async_copy(rows_v.at[j], acc_sh.at[si_v.at[j]], sems,
                                add=True) for j in range(8)]
        for d in ds2:
            d.wait()
        return c
    lax.fori_loop(0, 98, body, 0)
    plsc.subcore_barrier()
    pltpu.sync_copy(acc_sh.at[pl.ds(sid * 6400, 6400)],
                    acc2.at[cid].at[pl.ds(sid * 6400, 6400)])


def _pair_body(pos0r, pos1r, h1, ga, gb, idx_v, rows_v, sem):
    wid = lax.axis_index("s") * _NC + lax.axis_index("c")
    for idxr, outr in ((pos0r, ga), (pos1r, gb)):
        def body(g, c, idxr=idxr, outr=outr):
            rbase = wid * 25 + g * 5
            pltpu.sync_copy(idxr.at[pl.ds(rbase, 5)], idx_v)
            ds1 = [pltpu.async_copy(h1.at[idx_v.at[j]], rows_v.at[j], sem)
                   for j in range(5)]
            for d in ds1:
                d.wait()
            ds2 = [pltpu.async_copy(rows_v.at[j],
                                    outr.at[pl.ds((rbase + j) * 128, 128)], sem)
                   for j in range(5)]
            for d in ds2:
                d.wait()
            return c
        lax.fori_loop(0, 5, body, 0)


def _final_body(ier, ior, h2, he, ho, idx_v, rows_v, sem):
    wid = lax.axis_index("s") * _NC + lax.axis_index("c")
    for idxr, outr in ((ier, he), (ior, ho)):
        base = wid * 13
        for lo, k in ((0, 7), (7, 6)):
            pltpu.sync_copy(idxr.at[pl.ds(base + lo, k)], idx_v.at[pl.ds(0, k)])
            ds1 = [pltpu.async_copy(h2.at[idx_v.at[j]], rows_v.at[j], sem)
                   for j in range(k)]
            for d in ds1:
                d.wait()
            ds2 = [pltpu.async_copy(rows_v.at[j],
                                    outr.at[pl.ds((base + lo + j) * 128, 128)], sem)
                   for j in range(k)]
            for d in ds2:
                d.wait()


def _t1_body(emb_ref, w_ref, o_ref):
    o_ref[...] = jnp.dot(emb_ref[...], w_ref[...], precision=HI)


def _t3_body(g_ref, deg_ref, o_ref):
    o_ref[...] = g_ref[...] * _rs(deg_ref[...])


def _t4_body(accp_ref, hw_ref, deg_ref, b_ref, o_ref):
    acc = accp_ref[0] + accp_ref[1] - hw_ref[...]
    o_ref[...] = jax.nn.relu(_rs(deg_ref[...]) * acc + b_ref[...])


def _t5_body(ga_ref, gb_ref, wa_ref, wb_ref, dega_ref, degb_ref, oa_ref, ob_ref):
    hp = ga_ref[...] * gb_ref[...]
    ha = jnp.dot(hp, wa_ref[...], precision=HI) * _rs(dega_ref[...])
    hb = jnp.dot(hp, wb_ref[...], precision=HI) * _rs(degb_ref[...])
    oa_ref[0] = ha[:, :_H]
    oa_ref[1] = ha[:, _H:]
    ob_ref[0] = hb[:, :_H]
    ob_ref[1] = hb[:, _H:]


def _t6_body(acca_ref, accb_ref, dega_ref, degb_ref, ba_ref, bb_ref, o_ref):
    acca = jnp.concatenate([acca_ref[0], acca_ref[1]], axis=1)
    accb = jnp.concatenate([accb_ref[0], accb_ref[1]], axis=1)
    o_ref[...] = (jax.nn.relu(_rs(dega_ref[...]) * acca + ba_ref[...])
                  + jax.nn.relu(_rs(degb_ref[...]) * accb + bb_ref[...]))


def _t7_body(he_ref, ho_ref, w_ref, b_ref, o_ref):
    o_ref[...] = jnp.sum(he_ref[...] * ho_ref[...] * w_ref[...], axis=1,
                         keepdims=True) + b_ref[...]


def _sds(shape, dtype=F32):
    return jax.ShapeDtypeStruct(shape, dtype)


_s1_call = pl.kernel(
    _count_gather_body,
    out_type=(_sds((2, _N2)), _sds((2, _P2)), _sds((2, _P2)), _sds((_N2, 32))),
    mesh=_MESH,
    compiler_params=_SC_PARAMS,
    scratch_types=[
        pltpu.VMEM_SHARED((_N2,), F32),
        pltpu.VMEM_SHARED((_P2,), F32),
        pltpu.VMEM_SHARED((_P2,), F32),
        pltpu.VMEM((128,), F32),
        pltpu.VMEM((6400,), F32),
        pltpu.VMEM((8, 128), I32),
        pltpu.VMEM((4, 128), I32),
        pltpu.VMEM((4, 128, 32), F32),
        pltpu.SemaphoreType.DMA,
    ])

_conv1_call = pl.kernel(
    _conv1_body,
    out_type=_sds((2, _N2, 32)),
    mesh=_MESH,
    compiler_params=_SC_PARAMS,
    scratch_types=[
        pltpu.VMEM_SHARED((_N2, 32), F32),
        pltpu.VMEM((8, 128), I32),
        pltpu.VMEM((8, 128), I32),
        pltpu.VMEM((8, 128, 32), F32),
        pltpu.SemaphoreType.DMA,
        pltpu.SemaphoreType.DMA,
    ])

_conv2_call = pl.kernel(
    _conv2_body,
    out_type=_sds((2, _P2, _H)),
    mesh=_MESH,
    compiler_params=_SC_PARAMS,
    scratch_types=[
        pltpu.VMEM_SHARED((_P2, _H), F32),
        pltpu.VMEM((8, 128), I32),
        pltpu.VMEM((8, 128), I32),
        pltpu.VMEM((8, 128, _H), F32),
        pltpu.SemaphoreType.DMA,
        pltpu.SemaphoreType.DMA,
    ])

_pair_call = pl.kernel(
    _pair_body,
    out_type=(_sds((_P2, 32)), _sds((_P2, 32))),
    mesh=_MESH,
    compiler_params=_SC_PARAMS,
    scratch_types=[
        pltpu.VMEM((5, 128), I32),
        pltpu.VMEM((5, 128, 32), F32),
        pltpu.SemaphoreType.DMA,
    ])

_final_call = pl.kernel(
    _final_body,
    out_type=(_sds((_F2, 32)), _sds((_F2, 32))),
    mesh=_MESH,
    compiler_params=_SC_PARAMS,
    scratch_types=[
        pltpu.VMEM((7, 128), I32),
        pltpu.VMEM((7, 128, 32), F32),
        pltpu.SemaphoreType.DMA,
    ])

_t1_call = pl.pallas_call(
    _t1_body, grid=(10,),
    in_specs=[pl.BlockSpec((1000, 128), lambda i: (i, 0)),
              pl.BlockSpec((128, 32), lambda i: (0, 0))],
    out_specs=pl.BlockSpec((1000, 32), lambda i: (i, 0)),
    out_shape=_sds((_N, 32)))

_t3_call = pl.pallas_call(
    _t3_body, grid=(10,),
    in_specs=[pl.BlockSpec((1024, 32), lambda i: (i, 0)),
              pl.BlockSpec((1024, 1), lambda i: (i, 0))],
    out_specs=pl.BlockSpec((1024, 32), lambda i: (i, 0)),
    out_shape=_sds((_N2, 32)))

_t4_call = pl.pallas_call(
    _t4_body, grid=(10,),
    in_specs=[pl.BlockSpec((2, 1024, 32), lambda i: (0, i, 0)),
              pl.BlockSpec((1024, 32), lambda i: (i, 0)),
              pl.BlockSpec((1024, 1), lambda i: (i, 0)),
              pl.BlockSpec((1, 32), lambda i: (0, 0))],
    out_specs=pl.BlockSpec((1024, 32), lambda i: (i, 0)),
    out_shape=_sds((_N2, 32)))

_t5_call = pl.pallas_call(
    _t5_body, grid=(100,),
    in_specs=[pl.BlockSpec((1024, 32), lambda i: (i, 0)),
              pl.BlockSpec((1024, 32), lambda i: (i, 0)),
              pl.BlockSpec((32, 32), lambda i: (0, 0)),
              pl.BlockSpec((32, 32), lambda i: (0, 0)),
              pl.BlockSpec((1024, 1), lambda i: (i, 0)),
              pl.BlockSpec((1024, 1), lambda i: (i, 0))],
    out_specs=(pl.BlockSpec((2, 1024, _H), lambda i: (0, i, 0)),
               pl.BlockSpec((2, 1024, _H), lambda i: (0, i, 0))),
    out_shape=(_sds((2, _P2, _H)), _sds((2, _P2, _H))))

_t6_call = pl.pallas_call(
    _t6_body, grid=(100,),
    in_specs=[pl.BlockSpec((2, 1024, _H), lambda i: (0, i, 0)),
              pl.BlockSpec((2, 1024, _H), lambda i: (0, i, 0)),
              pl.BlockSpec((1024, 1), lambda i: (i, 0)),
              pl.BlockSpec((1024, 1), lambda i: (i, 0)),
              pl.BlockSpec((1, 32), lambda i: (0, 0)),
              pl.BlockSpec((1, 32), lambda i: (0, 0))],
    out_specs=pl.BlockSpec((1024, 32), lambda i: (i, 0)),
    out_shape=_sds((_P2, 32)))

_t7_call = pl.pallas_call(
    _t7_body, grid=(52,),
    in_specs=[pl.BlockSpec((1024, 32), lambda i: (i, 0)),
              pl.BlockSpec((1024, 32), lambda i: (i, 0)),
              pl.BlockSpec((1, 32), lambda i: (0, 0)),
              pl.BlockSpec((1, 1), lambda i: (0, 0))],
    out_specs=pl.BlockSpec((1024, 1), lambda i: (i, 0)),
    out_shape=_sds((_F2, 1)))


def _pad_idx(a, n, v):
    return jnp.concatenate([a.astype(I32), jnp.full((n - a.shape[0],), v, I32)])


def kernel(x, edge1, pos, idx, ei2, emb, W1, b1, W2a, b2a, W2b, b2b, Wp, bp):
    s1r = _pad_idx(edge1[0], _E1PAD, _N).reshape(_E1R, 128)
    d1r = _pad_idx(edge1[1], _E1PAD, _N).reshape(_E1R, 128)
    s2r = _pad_idx(ei2[0], _E2PAD, _P).reshape(_E2R, 128)
    d2r = _pad_idx(ei2[1], _E2PAD, _P).reshape(_E2R, 128)
    xr = _pad_idx(x, _N2, 0).reshape(_NR, 128)
    pos0r = _pad_idx(pos[:, 0], _P2, 0).reshape(_PR, 128)
    pos1r = _pad_idx(pos[:, 1], _P2, 0).reshape(_PR, 128)
    idx2 = idx.astype(I32).reshape(_F, 2)
    ier = _pad_idx(idx2[:, 0], _F2, 0).reshape(_FR, 128)
    ior = _pad_idx(idx2[:, 1], _F2, 0).reshape(_FR, 128)

    embw = _t1_call(emb, W1)
    deg1p, degap, degbp, g1 = _s1_call(d1r, s2r, d2r, xr, embw)
    deg1c = (deg1p[0] + deg1p[1] + 1.0).reshape(_N2, 1)
    degac = (degap[0] + degap[1] + 1.0).reshape(_P2, 1)
    degbc = (degbp[0] + degbp[1] + 1.0).reshape(_P2, 1)

    hw1p = _t3_call(g1, deg1c)
    accp = _conv1_call(s1r, d1r, hw1p)
    h1 = _t4_call(accp, hw1p, deg1c, b1.reshape(1, 32))

    ga, gb = _pair_call(pos0r, pos1r, h1)
    hap2, hbp2 = _t5_call(ga, gb, W2a, W2b, degac, degbc)
    acca2 = _conv2_call(s2r, d2r, hap2)
    accb2 = _conv2_call(d2r, s2r, hbp2)
    h2 = _t6_call(acca2, accb2, degac, degbc, b2a.reshape(1, 32),
                  b2b.reshape(1, 32))

    he, ho = _final_call(ier, ior, h2)
    out = _t7_call(he, ho, Wp.reshape(1, 32), bp.reshape(1, 1))
    return out[:_F]


# speedup vs baseline: 44.0998x; 1.2304x over previous
import jax
import jax.numpy as jnp
from jax import lax
from jax.experimental import pallas as pl
from jax.experimental.pallas import tpu as pltpu
from jax.experimental.pallas import tpu_sc as plsc

F32 = jnp.float32
I32 = jnp.int32
HI = lax.Precision.HIGHEST

_N = 10000
_N2 = 10240
_NR = _N2 // 128
_E1PAD = 655360
_E1R = _E1PAD // 128
_P = 100000
_P2 = 102400
_PR = _P2 // 128
_E2PAD = 1605632
_E2R = _E2PAD // 128
_F = 50000
_F2 = 53248
_FR = _F2 // 128
_H = 16

_NC, _NS = 2, 16
_MESH = plsc.VectorSubcoreMesh(
    core_axis_name="c", subcore_axis_name="s", num_cores=_NC, num_subcores=_NS)
_SC_PARAMS = pltpu.CompilerParams(use_tc_tiling_on_sc=False)


def _rs(d):
    return lax.rsqrt(jnp.maximum(d, 1e-12))


def _count_gather_body(d1r, s2r, d2r, xr, embw,
                       deg1p, degap, degbp, g1,
                       deg1_sh, dega_sh, degb_sh,
                       ones_v, zero_v, idx_v, gidx_v, grow_v, sem, semx):
    cid = lax.axis_index("c")
    sid = lax.axis_index("s")

    for i in range(8):
        ones_v[pl.ds(i * 16, 16)] = jnp.ones((16,), F32)

    def _z(i, c):
        zero_v[pl.ds(i * 16, 16)] = jnp.zeros((16,), F32)
        return c
    lax.fori_loop(0, 400, _z, 0)

    pltpu.sync_copy(zero_v.at[pl.ds(0, 640)], deg1_sh.at[pl.ds(sid * 640, 640)])
    pltpu.sync_copy(zero_v, dega_sh.at[pl.ds(sid * 6400, 6400)])
    pltpu.sync_copy(zero_v, degb_sh.at[pl.ds(sid * 6400, 6400)])
    plsc.subcore_barrier()

    def _count(idx_hbm, acc_sh, nrows_tile):
        base = cid * (_NS * nrows_tile) + sid * nrows_tile
        ng = nrows_tile // 8

        def _fire(parity, g):
            for j in range(8):
                pltpu.async_copy(ones_v, acc_sh.at[idx_v.at[parity].at[j]], sem,
                                 add=True)

        def _drain(parity):
            for j in range(8):
                pltpu.make_async_copy(ones_v, acc_sh.at[idx_v.at[parity].at[j]],
                                      sem).wait()

        pltpu.sync_copy(idx_hbm.at[pl.ds(base, 8)], idx_v.at[0])
        _fire(0, 0)

        def body(g, c):
            p = lax.rem(g, 2)
            q = 1 - p
            pltpu.sync_copy(idx_hbm.at[pl.ds(base + (g + 1) * 8, 8)], idx_v.at[q])
            _drain(p)
            _fire(q, g + 1)
            return c
        lax.fori_loop(0, ng - 1, body, 0)
        _drain(lax.rem(ng - 1, 2))

    _count(d1r, deg1_sh, 160)
    _count(d2r, dega_sh, 392)
    _count(s2r, degb_sh, 392)

    wid = sid * _NC + cid

    @pl.when(wid < 20)
    def _():
        base = wid * 4
        pltpu.sync_copy(xr.at[pl.ds(base, 4)], gidx_v)
        descs = [pltpu.async_copy(embw.at[gidx_v.at[j]], grow_v.at[j], semx)
                 for j in range(4)]
        for d in descs:
            d.wait()
        descs = [pltpu.async_copy(grow_v.at[j], g1.at[pl.ds((base + j) * 128, 128)],
                                  semx) for j in range(4)]
        for d in descs:
            d.wait()

    plsc.subcore_barrier()
    pltpu.sync_copy(deg1_sh.at[pl.ds(sid * 640, 640)],
                    deg1p.at[cid].at[pl.ds(sid * 640, 640)])
    pltpu.sync_copy(dega_sh.at[pl.ds(sid * 6400, 6400)],
                    degap.at[cid].at[pl.ds(sid * 6400, 6400)])
    pltpu.sync_copy(degb_sh.at[pl.ds(sid * 6400, 6400)],
                    degbp.at[cid].at[pl.ds(sid * 6400, 6400)])


def _edge_pipeline(gir, sir, tbl, acc_sh, gi_v, si_v, rows_v, semg, sems,
                   base, ng, k):
    def _fire_g(p):
        for j in range(k):
            pltpu.async_copy(tbl.at[gi_v.at[p].at[j]], rows_v.at[p].at[j], semg)

    def _wait_g(p):
        for j in range(k):
            pltpu.make_async_copy(tbl.at[gi_v.at[p].at[j]], rows_v.at[p].at[j],
                                  semg).wait()

    def _fire_s(p):
        for j in range(k):
            pltpu.async_copy(rows_v.at[p].at[j], acc_sh.at[si_v.at[p].at[j]],
                             sems, add=True)

    def _wait_s(p):
        for j in range(k):
            pltpu.make_async_copy(rows_v.at[p].at[j], acc_sh.at[si_v.at[p].at[j]],
                                  sems).wait()

    pltpu.sync_copy(gir.at[pl.ds(base, k)], gi_v.at[0])
    pltpu.sync_copy(sir.at[pl.ds(base, k)], si_v.at[0])
    _fire_g(0)

    def body(g, c):
        p = lax.rem(g, 2)
        q = 1 - p
        pltpu.sync_copy(gir.at[pl.ds(base + (g + 1) * k, k)], gi_v.at[q])
        pltpu.sync_copy(sir.at[pl.ds(base + (g + 1) * k, k)], si_v.at[q])
        _wait_g(p)
        _fire_g(q)
        _fire_s(p)
        _wait_s(p)
        return c
    lax.fori_loop(0, ng - 1, body, 0)
    last = lax.rem(ng - 1, 2)
    _wait_g(last)
    _fire_s(last)
    _wait_s(last)


def _conv1_body(s1r, d1r, hw1p, accp,
                acc_sh, si_v, di_v, rows_v, semg, sems):
    cid = lax.axis_index("c")
    sid = lax.axis_index("s")
    pltpu.sync_copy(hw1p.at[pl.ds(sid * 640, 640)], acc_sh.at[pl.ds(sid * 640, 640)])
    plsc.subcore_barrier()
    base = cid * (_NS * 160) + sid * 160
    _edge_pipeline(s1r, d1r, hw1p, acc_sh, si_v, di_v, rows_v, semg, sems,
                   base, 20, 8)
    plsc.subcore_barrier()
    pltpu.sync_copy(acc_sh.at[pl.ds(sid * 640, 640)],
                    accp.at[cid].at[pl.ds(sid * 640, 640)])


def _conv2_body(s2r, d2r, tbla, tblb, acca, accb,
                acc_sh, gi_v, si_v, rows_v, semg, sems):
    cid = lax.axis_index("c")
    sid = lax.axis_index("s")
    base = sid * 784
    for gir, sir, tbl, out in ((s2r, d2r, tbla, acca), (d2r, s2r, tblb, accb)):
        pltpu.sync_copy(tbl.at[cid].at[pl.ds(sid * 6400, 6400)],
                        acc_sh.at[pl.ds(sid * 6400, 6400)])
        plsc.subcore_barrier()
        _edge_pipeline(gir, sir, tbl.at[cid], acc_sh, gi_v, si_v, rows_v,
                       semg, sems, base, 196, 4)
        plsc.subcore_barrier()
        pltpu.sync_copy(acc_sh.at[pl.ds(sid * 6400, 6400)],
                        out.at[cid].at[pl.ds(sid * 6400, 6400)])
        plsc.subcore_barrier()


def _gather_pipeline(idxr, tbl, outr, idx_v, rows_v, semg, semw, base, ng, k):
    def _fire_g(p, off):
        for j in range(k):
            pltpu.async_copy(tbl.at[idx_v.at[p].at[j]], rows_v.at[p].at[j], semg)

    def _wait_g(p):
        for j in range(k):
            pltpu.make_async_copy(tbl.at[idx_v.at[p].at[j]], rows_v.at[p].at[j],
                                  semg).wait()

    def _fire_w(p, grp):
        for j in range(k):
            pltpu.async_copy(rows_v.at[p].at[j],
                             outr.at[pl.ds((base + grp * k + j) * 128, 128)],
                             semw)

    def _wait_w(p, grp):
        for j in range(k):
            pltpu.make_async_copy(rows_v.at[p].at[j],
                                  outr.at[pl.ds((base + grp * k + j) * 128, 128)],
                                  semw).wait()

    pltpu.sync_copy(idxr.at[pl.ds(base, k)], idx_v.at[0])
    _fire_g(0, 0)

    def body(g, c):
        p = lax.rem(g, 2)
        q = 1 - p
        pltpu.sync_copy(idxr.at[pl.ds(base + (g + 1) * k, k)], idx_v.at[q])
        _wait_g(p)
        _fire_g(q, g + 1)
        _fire_w(p, g)
        _wait_w(p, g)
        return c
    lax.fori_loop(0, ng - 1, body, 0)
    last = lax.rem(ng - 1, 2)
    _wait_g(last)
    _fire_w(last, ng - 1)
    _wait_w(last, ng - 1)


def _pair_body(pos0r, pos1r, h1, ga, gb, idx_v, rows_v, semg, semw):
    wid = lax.axis_index("s") * _NC + lax.axis_index("c")
    base = wid * 25
    _gather_pipeline(pos0r, h1, ga, idx_v, rows_v, semg, semw, base, 5, 5)
    _gather_pipeline(pos1r, h1, gb, idx_v, rows_v, semg, semw, base, 5, 5)


def _final_body(ier, ior, h2, he, ho, idx_v, rows_v, semg, semw):
    wid = lax.axis_index("s") * _NC + lax.axis_index("c")
    base = wid * 13
    for idxr, outr in ((ier, he), (ior, ho)):
        _gather_pipeline(idxr, h2, outr, idx_v, rows_v, semg, semw, base, 2, 6)
        pltpu.sync_copy(idxr.at[pl.ds(base + 12, 1)], idx_v.at[0].at[pl.ds(0, 1)])
        pltpu.async_copy(h2.at[idx_v.at[0].at[0]], rows_v.at[0].at[0], semg).wait()
        pltpu.async_copy(rows_v.at[0].at[0],
                         outr.at[pl.ds((base + 12) * 128, 128)], semw).wait()


def _t1_body(emb_ref, w_ref, o_ref):
    o_ref[...] = jnp.dot(emb_ref[...], w_ref[...], precision=HI)


def _t3_body(g_ref, deg_ref, o_ref):
    o_ref[...] = g_ref[...] * _rs(deg_ref[...])


def _t4_body(accp_ref, hw_ref, deg_ref, b_ref, o_ref):
    acc = accp_ref[0] + accp_ref[1] - hw_ref[...]
    o_ref[...] = jax.nn.relu(_rs(deg_ref[...]) * acc + b_ref[...])


def _t5_body(ga_ref, gb_ref, wa_ref, wb_ref, dega_ref, degb_ref, oa_ref, ob_ref):
    hp = ga_ref[...] * gb_ref[...]
    ha = jnp.dot(hp, wa_ref[...], precision=HI) * _rs(dega_ref[...])
    hb = jnp.dot(hp, wb_ref[...], precision=HI) * _rs(degb_ref[...])
    oa_ref[0] = ha[:, :_H]
    oa_ref[1] = ha[:, _H:]
    ob_ref[0] = hb[:, :_H]
    ob_ref[1] = hb[:, _H:]


def _t6_body(acca_ref, accb_ref, dega_ref, degb_ref, ba_ref, bb_ref, o_ref):
    acca = jnp.concatenate([acca_ref[0], acca_ref[1]], axis=1)
    accb = jnp.concatenate([accb_ref[0], accb_ref[1]], axis=1)
    o_ref[...] = (jax.nn.relu(_rs(dega_ref[...]) * acca + ba_ref[...])
                  + jax.nn.relu(_rs(degb_ref[...]) * accb + bb_ref[...]))


def _t7_body(he_ref, ho_ref, w_ref, b_ref, o_ref):
    o_ref[...] = jnp.sum(he_ref[...] * ho_ref[...] * w_ref[...], axis=1,
                         keepdims=True) + b_ref[...]


def _sds(shape, dtype=F32):
    return jax.ShapeDtypeStruct(shape, dtype)


_s1_call = pl.kernel(
    _count_gather_body,
    out_type=(_sds((2, _N2)), _sds((2, _P2)), _sds((2, _P2)), _sds((_N2, 32))),
    mesh=_MESH,
    compiler_params=_SC_PARAMS,
    scratch_types=[
        pltpu.VMEM_SHARED((_N2,), F32),
        pltpu.VMEM_SHARED((_P2,), F32),
        pltpu.VMEM_SHARED((_P2,), F32),
        pltpu.VMEM((128,), F32),
        pltpu.VMEM((6400,), F32),
        pltpu.VMEM((2, 8, 128), I32),
        pltpu.VMEM((4, 128), I32),
        pltpu.VMEM((4, 128, 32), F32),
        pltpu.SemaphoreType.DMA,
        pltpu.SemaphoreType.DMA,
    ])

_conv1_call = pl.kernel(
    _conv1_body,
    out_type=_sds((2, _N2, 32)),
    mesh=_MESH,
    compiler_params=_SC_PARAMS,
    scratch_types=[
        pltpu.VMEM_SHARED((_N2, 32), F32),
        pltpu.VMEM((2, 8, 128), I32),
        pltpu.VMEM((2, 8, 128), I32),
        pltpu.VMEM((2, 8, 128, 32), F32),
        pltpu.SemaphoreType.DMA,
        pltpu.SemaphoreType.DMA,
    ])

_conv2_call = pl.kernel(
    _conv2_body,
    out_type=(_sds((2, _P2, _H)), _sds((2, _P2, _H))),
    mesh=_MESH,
    compiler_params=_SC_PARAMS,
    scratch_types=[
        pltpu.VMEM_SHARED((_P2, _H), F32),
        pltpu.VMEM((2, 4, 128), I32),
        pltpu.VMEM((2, 4, 128), I32),
        pltpu.VMEM((2, 4, 128, _H), F32),
        pltpu.SemaphoreType.DMA,
        pltpu.SemaphoreType.DMA,
    ])

_pair_call = pl.kernel(
    _pair_body,
    out_type=(_sds((_P2, 32)), _sds((_P2, 32))),
    mesh=_MESH,
    compiler_params=_SC_PARAMS,
    scratch_types=[
        pltpu.VMEM((2, 5, 128), I32),
        pltpu.VMEM((2, 5, 128, 32), F32),
        pltpu.SemaphoreType.DMA,
        pltpu.SemaphoreType.DMA,
    ])

_final_call = pl.kernel(
    _final_body,
    out_type=(_sds((_F2, 32)), _sds((_F2, 32))),
    mesh=_MESH,
    compiler_params=_SC_PARAMS,
    scratch_types=[
        pltpu.VMEM((2, 6, 128), I32),
        pltpu.VMEM((2, 6, 128, 32), F32),
        pltpu.SemaphoreType.DMA,
        pltpu.SemaphoreType.DMA,
    ])

_t1_call = pl.pallas_call(
    _t1_body, grid=(10,),
    in_specs=[pl.BlockSpec((1000, 128), lambda i: (i, 0)),
              pl.BlockSpec((128, 32), lambda i: (0, 0))],
    out_specs=pl.BlockSpec((1000, 32), lambda i: (i, 0)),
    out_shape=_sds((_N, 32)))

_t3_call = pl.pallas_call(
    _t3_body, grid=(10,),
    in_specs=[pl.BlockSpec((1024, 32), lambda i: (i, 0)),
              pl.BlockSpec((1024, 1), lambda i: (i, 0))],
    out_specs=pl.BlockSpec((1024, 32), lambda i: (i, 0)),
    out_shape=_sds((_N2, 32)))

_t4_call = pl.pallas_call(
    _t4_body, grid=(10,),
    in_specs=[pl.BlockSpec((2, 1024, 32), lambda i: (0, i, 0)),
              pl.BlockSpec((1024, 32), lambda i: (i, 0)),
              pl.BlockSpec((1024, 1), lambda i: (i, 0)),
              pl.BlockSpec((1, 32), lambda i: (0, 0))],
    out_specs=pl.BlockSpec((1024, 32), lambda i: (i, 0)),
    out_shape=_sds((_N2, 32)))

_t5_call = pl.pallas_call(
    _t5_body, grid=(100,),
    in_specs=[pl.BlockSpec((1024, 32), lambda i: (i, 0)),
              pl.BlockSpec((1024, 32), lambda i: (i, 0)),
              pl.BlockSpec((32, 32), lambda i: (0, 0)),
              pl.BlockSpec((32, 32), lambda i: (0, 0)),
              pl.BlockSpec((1024, 1), lambda i: (i, 0)),
              pl.BlockSpec((1024, 1), lambda i: (i, 0))],
    out_specs=(pl.BlockSpec((2, 1024, _H), lambda i: (0, i, 0)),
               pl.BlockSpec((2, 1024, _H), lambda i: (0, i, 0))),
    out_shape=(_sds((2, _P2, _H)), _sds((2, _P2, _H))))

_t6_call = pl.pallas_call(
    _t6_body, grid=(100,),
    in_specs=[pl.BlockSpec((2, 1024, _H), lambda i: (0, i, 0)),
              pl.BlockSpec((2, 1024, _H), lambda i: (0, i, 0)),
              pl.BlockSpec((1024, 1), lambda i: (i, 0)),
              pl.BlockSpec((1024, 1), lambda i: (i, 0)),
              pl.BlockSpec((1, 32), lambda i: (0, 0)),
              pl.BlockSpec((1, 32), lambda i: (0, 0))],
    out_specs=pl.BlockSpec((1024, 32), lambda i: (i, 0)),
    out_shape=_sds((_P2, 32)))

_t7_call = pl.pallas_call(
    _t7_body, grid=(52,),
    in_specs=[pl.BlockSpec((1024, 32), lambda i: (i, 0)),
              pl.BlockSpec((1024, 32), lambda i: (i, 0)),
              pl.BlockSpec((1, 32), lambda i: (0, 0)),
              pl.BlockSpec((1, 1), lambda i: (0, 0))],
    out_specs=pl.BlockSpec((1024, 1), lambda i: (i, 0)),
    out_shape=_sds((_F2, 1)))


def _pad_spread(a, n, base, mod):
    fill = base + (jnp.arange(n - a.shape[0], dtype=I32) % mod)
    return jnp.concatenate([a.astype(I32), fill])


def kernel(x, edge1, pos, idx, ei2, emb, W1, b1, W2a, b2a, W2b, b2b, Wp, bp):
    s1r = _pad_spread(edge1[0], _E1PAD, _N, _N2 - _N).reshape(_E1R, 128)
    d1r = _pad_spread(edge1[1], _E1PAD, _N, _N2 - _N).reshape(_E1R, 128)
    s2r = _pad_spread(ei2[0], _E2PAD, _P, _P2 - _P).reshape(_E2R, 128)
    d2r = _pad_spread(ei2[1], _E2PAD, _P, _P2 - _P).reshape(_E2R, 128)
    xr = _pad_spread(x, _N2, 0, _N).reshape(_NR, 128)
    pos0r = _pad_spread(pos[:, 0], _P2, 0, _N).reshape(_PR, 128)
    pos1r = _pad_spread(pos[:, 1], _P2, 0, _N).reshape(_PR, 128)
    idx2 = idx.astype(I32).reshape(_F, 2)
    ier = _pad_spread(idx2[:, 0], _F2, 0, _P).reshape(_FR, 128)
    ior = _pad_spread(idx2[:, 1], _F2, 0, _P).reshape(_FR, 128)

    embw = _t1_call(emb, W1)
    deg1p, degap, degbp, g1 = _s1_call(d1r, s2r, d2r, xr, embw)
    deg1c = (deg1p[0] + deg1p[1] + 1.0).reshape(_N2, 1)
    degac = (degap[0] + degap[1] + 1.0).reshape(_P2, 1)
    degbc = (degbp[0] + degbp[1] + 1.0).reshape(_P2, 1)

    hw1p = _t3_call(g1, deg1c)
    accp = _conv1_call(s1r, d1r, hw1p)
    h1 = _t4_call(accp, hw1p, deg1c, b1.reshape(1, 32))

    ga, gb = _pair_call(pos0r, pos1r, h1)
    hap2, hbp2 = _t5_call(ga, gb, W2a, W2b, degac, degbc)
    acca2, accb2 = _conv2_call(s2r, d2r, hap2, hbp2)
    h2 = _t6_call(acca2, accb2, degac, degbc, b2a.reshape(1, 32),
                  b2b.reshape(1, 32))

    he, ho = _final_call(ier, ior, h2)
    out = _t7_call(he, ho, Wp.reshape(1, 32), bp.reshape(1, 1))
    return out[:_F]


# speedup vs baseline: 47.1514x; 1.0692x over previous
import jax
import jax.numpy as jnp
from jax import lax
from jax.experimental import pallas as pl
from jax.experimental.pallas import tpu as pltpu
from jax.experimental.pallas import tpu_sc as plsc

F32 = jnp.float32
I32 = jnp.int32
HI = lax.Precision.HIGHEST

_N = 10000
_N2 = 10240
_NR = _N2 // 128
_E1PAD = 655360
_E1R = _E1PAD // 128
_P = 100000
_P2 = 102400
_PR = _P2 // 128
_E2PAD = 1605632
_E2R = _E2PAD // 128
_F = 50000
_F2 = 53248
_FR = _F2 // 128
_H = 16

_NC, _NS = 2, 16
_MESH = plsc.VectorSubcoreMesh(
    core_axis_name="c", subcore_axis_name="s", num_cores=_NC, num_subcores=_NS)
_SC_PARAMS = pltpu.CompilerParams(use_tc_tiling_on_sc=False)


def _rs(d):
    return lax.rsqrt(jnp.maximum(d, 1e-12))


def _count_gather_body(d1r, s2r, d2r, xr, emb,
                       deg1p, degap, degbp, g0,
                       deg1_sh, dega_sh, degb_sh,
                       ones_v, zero_v, idx_v, gidx_v, grow_v, sem, semx):
    cid = lax.axis_index("c")
    sid = lax.axis_index("s")

    for i in range(8):
        ones_v[pl.ds(i * 16, 16)] = jnp.ones((16,), F32)

    def _z(i, c):
        zero_v[pl.ds(i * 16, 16)] = jnp.zeros((16,), F32)
        return c
    lax.fori_loop(0, 400, _z, 0)

    pltpu.sync_copy(zero_v.at[pl.ds(0, 640)], deg1_sh.at[pl.ds(sid * 640, 640)])
    pltpu.sync_copy(zero_v, dega_sh.at[pl.ds(sid * 6400, 6400)])
    pltpu.sync_copy(zero_v, degb_sh.at[pl.ds(sid * 6400, 6400)])
    plsc.subcore_barrier()

    def _count(idx_hbm, acc_sh, nrows_tile):
        base = cid * (_NS * nrows_tile) + sid * nrows_tile
        ng = nrows_tile // 8

        def _fire(parity, g):
            for j in range(8):
                pltpu.async_copy(ones_v, acc_sh.at[idx_v.at[parity].at[j]], sem,
                                 add=True)

        def _drain(parity):
            for j in range(8):
                pltpu.make_async_copy(ones_v, acc_sh.at[idx_v.at[parity].at[j]],
                                      sem).wait()

        pltpu.sync_copy(idx_hbm.at[pl.ds(base, 8)], idx_v.at[0])
        _fire(0, 0)

        def body(g, c):
            p = lax.rem(g, 2)
            q = 1 - p
            pltpu.sync_copy(idx_hbm.at[pl.ds(base + (g + 1) * 8, 8)], idx_v.at[q])
            _drain(p)
            _fire(q, g + 1)
            return c
        lax.fori_loop(0, ng - 1, body, 0)
        _drain(lax.rem(ng - 1, 2))

    _count(d1r, deg1_sh, 160)
    _count(d2r, dega_sh, 392)
    _count(s2r, degb_sh, 392)

    wid = sid * _NC + cid

    @pl.when(wid < 20)
    def _():
        base = wid * 4
        pltpu.sync_copy(xr.at[pl.ds(base, 4)], gidx_v)
        descs = [pltpu.async_copy(emb.at[gidx_v.at[j]], grow_v.at[j], semx)
                 for j in range(4)]
        for d in descs:
            d.wait()
        descs = [pltpu.async_copy(grow_v.at[j], g0.at[pl.ds((base + j) * 128, 128)],
                                  semx) for j in range(4)]
        for d in descs:
            d.wait()

    plsc.subcore_barrier()
    pltpu.sync_copy(deg1_sh.at[pl.ds(sid * 640, 640)],
                    deg1p.at[cid].at[pl.ds(sid * 640, 640)])
    pltpu.sync_copy(dega_sh.at[pl.ds(sid * 6400, 6400)],
                    degap.at[cid].at[pl.ds(sid * 6400, 6400)])
    pltpu.sync_copy(degb_sh.at[pl.ds(sid * 6400, 6400)],
                    degbp.at[cid].at[pl.ds(sid * 6400, 6400)])


def _edge_pipeline(gir, sir, tbl, acc_sh, gi_v, si_v, rows_v, semg, sems,
                   semi, base, ng, k):
    def _fire_g(rp, ip):
        for j in range(k):
            pltpu.async_copy(tbl.at[gi_v.at[ip].at[j]], rows_v.at[rp].at[j], semg)

    def _wait_g(rp, ip):
        for j in range(k):
            pltpu.make_async_copy(tbl.at[gi_v.at[ip].at[j]], rows_v.at[rp].at[j],
                                  semg).wait()

    def _fire_s(rp, ip):
        for j in range(k):
            pltpu.async_copy(rows_v.at[rp].at[j], acc_sh.at[si_v.at[ip].at[j]],
                             sems, add=True)

    def _wait_s(rp, ip):
        for j in range(k):
            pltpu.make_async_copy(rows_v.at[rp].at[j], acc_sh.at[si_v.at[ip].at[j]],
                                  sems).wait()

    def _fire_i(g, slot):
        pltpu.async_copy(gir.at[pl.ds(base + g * k, k)], gi_v.at[slot], semi)
        pltpu.async_copy(sir.at[pl.ds(base + g * k, k)], si_v.at[slot], semi)

    def _wait_i(g, slot):
        pltpu.make_async_copy(gir.at[pl.ds(base + g * k, k)], gi_v.at[slot],
                              semi).wait()
        pltpu.make_async_copy(sir.at[pl.ds(base + g * k, k)], si_v.at[slot],
                              semi).wait()

    pltpu.sync_copy(gir.at[pl.ds(base, k)], gi_v.at[0])
    pltpu.sync_copy(sir.at[pl.ds(base, k)], si_v.at[0])
    _fire_g(0, 0)
    _fire_i(1, 1)

    def body(g, c):
        rp = lax.rem(g, 2)
        rq = 1 - rp
        ip = lax.rem(g, 3)
        iq = lax.rem(g + 1, 3)
        ir = lax.rem(g + 2, 3)
        _wait_i(g + 1, iq)
        _wait_g(rp, ip)
        _fire_g(rq, iq)

        @pl.when(g + 2 <= ng - 1)
        def _():
            _fire_i(g + 2, ir)
        _fire_s(rp, ip)
        _wait_s(rp, ip)
        return c
    lax.fori_loop(0, ng - 1, body, 0)
    last = ng - 1
    _wait_g(lax.rem(last, 2), lax.rem(last, 3))
    _fire_s(lax.rem(last, 2), lax.rem(last, 3))
    _wait_s(lax.rem(last, 2), lax.rem(last, 3))


def _conv1_body(s1r, d1r, hw1p, accp,
                acc_sh, si_v, di_v, rows_v, semg, sems, semi):
    cid = lax.axis_index("c")
    sid = lax.axis_index("s")
    pltpu.sync_copy(hw1p.at[pl.ds(sid * 640, 640)], acc_sh.at[pl.ds(sid * 640, 640)])
    plsc.subcore_barrier()
    base = cid * (_NS * 160) + sid * 160
    _edge_pipeline(s1r, d1r, hw1p, acc_sh, si_v, di_v, rows_v, semg, sems,
                   semi, base, 20, 8)
    plsc.subcore_barrier()
    pltpu.sync_copy(acc_sh.at[pl.ds(sid * 640, 640)],
                    accp.at[cid].at[pl.ds(sid * 640, 640)])


def _conv2_body(s2r, d2r, tbla, tblb, acca, accb,
                acc_sh, gi_v, si_v, rows_v, semg, sems, semi):
    cid = lax.axis_index("c")
    sid = lax.axis_index("s")
    base = sid * 784
    for gir, sir, tbl, out in ((s2r, d2r, tbla, acca), (d2r, s2r, tblb, accb)):
        pltpu.sync_copy(tbl.at[cid].at[pl.ds(sid * 6400, 6400)],
                        acc_sh.at[pl.ds(sid * 6400, 6400)])
        plsc.subcore_barrier()
        _edge_pipeline(gir, sir, tbl.at[cid], acc_sh, gi_v, si_v, rows_v,
                       semg, sems, semi, base, 196, 4)
        plsc.subcore_barrier()
        pltpu.sync_copy(acc_sh.at[pl.ds(sid * 6400, 6400)],
                        out.at[cid].at[pl.ds(sid * 6400, 6400)])
        plsc.subcore_barrier()


def _gather_pipeline(idxr, tbl, outr, idx_v, rows_v, semg, semw, base, ng, k):
    def _fire_g(p, off):
        for j in range(k):
            pltpu.async_copy(tbl.at[idx_v.at[p].at[j]], rows_v.at[p].at[j], semg)

    def _wait_g(p):
        for j in range(k):
            pltpu.make_async_copy(tbl.at[idx_v.at[p].at[j]], rows_v.at[p].at[j],
                                  semg).wait()

    def _fire_w(p, grp):
        for j in range(k):
            pltpu.async_copy(rows_v.at[p].at[j],
                             outr.at[pl.ds((base + grp * k + j) * 128, 128)],
                             semw)

    def _wait_w(p, grp):
        for j in range(k):
            pltpu.make_async_copy(rows_v.at[p].at[j],
                                  outr.at[pl.ds((base + grp * k + j) * 128, 128)],
                                  semw).wait()

    pltpu.sync_copy(idxr.at[pl.ds(base, k)], idx_v.at[0])
    _fire_g(0, 0)

    def body(g, c):
        p = lax.rem(g, 2)
        q = 1 - p
        pltpu.sync_copy(idxr.at[pl.ds(base + (g + 1) * k, k)], idx_v.at[q])
        _wait_g(p)
        _fire_g(q, g + 1)
        _fire_w(p, g)
        _wait_w(p, g)
        return c
    lax.fori_loop(0, ng - 1, body, 0)
    last = lax.rem(ng - 1, 2)
    _wait_g(last)
    _fire_w(last, ng - 1)
    _wait_w(last, ng - 1)


def _pair_body(pos0r, pos1r, h1, ga, gb, idx_v, rows_v, semg, semw):
    wid = lax.axis_index("s") * _NC + lax.axis_index("c")
    base = wid * 25
    _gather_pipeline(pos0r, h1, ga, idx_v, rows_v, semg, semw, base, 5, 5)
    _gather_pipeline(pos1r, h1, gb, idx_v, rows_v, semg, semw, base, 5, 5)


def _final_body(ier, ior, h2, he, ho, idx_v, rows_v, semg, semw):
    wid = lax.axis_index("s") * _NC + lax.axis_index("c")
    base = wid * 13
    for idxr, outr in ((ier, he), (ior, ho)):
        _gather_pipeline(idxr, h2, outr, idx_v, rows_v, semg, semw, base, 2, 6)
        pltpu.sync_copy(idxr.at[pl.ds(base + 12, 1)], idx_v.at[0].at[pl.ds(0, 1)])
        pltpu.async_copy(h2.at[idx_v.at[0].at[0]], rows_v.at[0].at[0], semg).wait()
        pltpu.async_copy(rows_v.at[0].at[0],
                         outr.at[pl.ds((base + 12) * 128, 128)], semw).wait()


def _t1_body(g0_ref, w_ref, deg_ref, o_ref):
    o_ref[...] = jnp.dot(g0_ref[...], w_ref[...], precision=HI) * _rs(deg_ref[...])


def _t4_body(accp_ref, hw_ref, deg_ref, b_ref, o_ref):
    acc = accp_ref[0] + accp_ref[1] - hw_ref[...]
    o_ref[...] = jax.nn.relu(_rs(deg_ref[...]) * acc + b_ref[...])


def _t5_body(ga_ref, gb_ref, wa_ref, wb_ref, dega_ref, degb_ref, oa_ref, ob_ref):
    hp = ga_ref[...] * gb_ref[...]
    ha = jnp.dot(hp, wa_ref[...], precision=HI) * _rs(dega_ref[...])
    hb = jnp.dot(hp, wb_ref[...], precision=HI) * _rs(degb_ref[...])
    oa_ref[0] = ha[:, :_H]
    oa_ref[1] = ha[:, _H:]
    ob_ref[0] = hb[:, :_H]
    ob_ref[1] = hb[:, _H:]


def _t6_body(acca_ref, accb_ref, dega_ref, degb_ref, ba_ref, bb_ref, o_ref):
    acca = jnp.concatenate([acca_ref[0], acca_ref[1]], axis=1)
    accb = jnp.concatenate([accb_ref[0], accb_ref[1]], axis=1)
    o_ref[...] = (jax.nn.relu(_rs(dega_ref[...]) * acca + ba_ref[...])
                  + jax.nn.relu(_rs(degb_ref[...]) * accb + bb_ref[...]))


def _t7_body(he_ref, ho_ref, w_ref, b_ref, o_ref):
    o_ref[...] = jnp.sum(he_ref[...] * ho_ref[...] * w_ref[...], axis=1,
                         keepdims=True) + b_ref[...]


def _sds(shape, dtype=F32):
    return jax.ShapeDtypeStruct(shape, dtype)


_s1_call = pl.kernel(
    _count_gather_body,
    out_type=(_sds((2, _N2)), _sds((2, _P2)), _sds((2, _P2)), _sds((_N2, 128))),
    mesh=_MESH,
    compiler_params=_SC_PARAMS,
    scratch_types=[
        pltpu.VMEM_SHARED((_N2,), F32),
        pltpu.VMEM_SHARED((_P2,), F32),
        pltpu.VMEM_SHARED((_P2,), F32),
        pltpu.VMEM((128,), F32),
        pltpu.VMEM((6400,), F32),
        pltpu.VMEM((2, 8, 128), I32),
        pltpu.VMEM((4, 128), I32),
        pltpu.VMEM((4, 128, 128), F32),
        pltpu.SemaphoreType.DMA,
        pltpu.SemaphoreType.DMA,
    ])

_conv1_call = pl.kernel(
    _conv1_body,
    out_type=_sds((2, _N2, 32)),
    mesh=_MESH,
    compiler_params=_SC_PARAMS,
    scratch_types=[
        pltpu.VMEM_SHARED((_N2, 32), F32),
        pltpu.VMEM((3, 8, 128), I32),
        pltpu.VMEM((3, 8, 128), I32),
        pltpu.VMEM((2, 8, 128, 32), F32),
        pltpu.SemaphoreType.DMA,
        pltpu.SemaphoreType.DMA,
        pltpu.SemaphoreType.DMA,
    ])

_conv2_call = pl.kernel(
    _conv2_body,
    out_type=(_sds((2, _P2, _H)), _sds((2, _P2, _H))),
    mesh=_MESH,
    compiler_params=_SC_PARAMS,
    scratch_types=[
        pltpu.VMEM_SHARED((_P2, _H), F32),
        pltpu.VMEM((3, 4, 128), I32),
        pltpu.VMEM((3, 4, 128), I32),
        pltpu.VMEM((2, 4, 128, _H), F32),
        pltpu.SemaphoreType.DMA,
        pltpu.SemaphoreType.DMA,
        pltpu.SemaphoreType.DMA,
    ])

_pair_call = pl.kernel(
    _pair_body,
    out_type=(_sds((_P2, 32)), _sds((_P2, 32))),
    mesh=_MESH,
    compiler_params=_SC_PARAMS,
    scratch_types=[
        pltpu.VMEM((2, 5, 128), I32),
        pltpu.VMEM((2, 5, 128, 32), F32),
        pltpu.SemaphoreType.DMA,
        pltpu.SemaphoreType.DMA,
    ])

_final_call = pl.kernel(
    _final_body,
    out_type=(_sds((_F2, 32)), _sds((_F2, 32))),
    mesh=_MESH,
    compiler_params=_SC_PARAMS,
    scratch_types=[
        pltpu.VMEM((2, 6, 128), I32),
        pltpu.VMEM((2, 6, 128, 32), F32),
        pltpu.SemaphoreType.DMA,
        pltpu.SemaphoreType.DMA,
    ])

_t1_call = pl.pallas_call(
    _t1_body, grid=(10,),
    in_specs=[pl.BlockSpec((1024, 128), lambda i: (i, 0)),
              pl.BlockSpec((128, 32), lambda i: (0, 0)),
              pl.BlockSpec((1024, 1), lambda i: (i, 0))],
    out_specs=pl.BlockSpec((1024, 32), lambda i: (i, 0)),
    out_shape=_sds((_N2, 32)))

_t4_call = pl.pallas_call(
    _t4_body, grid=(10,),
    in_specs=[pl.BlockSpec((2, 1024, 32), lambda i: (0, i, 0)),
              pl.BlockSpec((1024, 32), lambda i: (i, 0)),
              pl.BlockSpec((1024, 1), lambda i: (i, 0)),
              pl.BlockSpec((1, 32), lambda i: (0, 0))],
    out_specs=pl.BlockSpec((1024, 32), lambda i: (i, 0)),
    out_shape=_sds((_N2, 32)))

_t5_call = pl.pallas_call(
    _t5_body, grid=(100,),
    in_specs=[pl.BlockSpec((1024, 32), lambda i: (i, 0)),
              pl.BlockSpec((1024, 32), lambda i: (i, 0)),
              pl.BlockSpec((32, 32), lambda i: (0, 0)),
              pl.BlockSpec((32, 32), lambda i: (0, 0)),
              pl.BlockSpec((1024, 1), lambda i: (i, 0)),
              pl.BlockSpec((1024, 1), lambda i: (i, 0))],
    out_specs=(pl.BlockSpec((2, 1024, _H), lambda i: (0, i, 0)),
               pl.BlockSpec((2, 1024, _H), lambda i: (0, i, 0))),
    out_shape=(_sds((2, _P2, _H)), _sds((2, _P2, _H))))

_t6_call = pl.pallas_call(
    _t6_body, grid=(100,),
    in_specs=[pl.BlockSpec((2, 1024, _H), lambda i: (0, i, 0)),
              pl.BlockSpec((2, 1024, _H), lambda i: (0, i, 0)),
              pl.BlockSpec((1024, 1), lambda i: (i, 0)),
              pl.BlockSpec((1024, 1), lambda i: (i, 0)),
              pl.BlockSpec((1, 32), lambda i: (0, 0)),
              pl.BlockSpec((1, 32), lambda i: (0, 0))],
    out_specs=pl.BlockSpec((1024, 32), lambda i: (i, 0)),
    out_shape=_sds((_P2, 32)))

_t7_call = pl.pallas_call(
    _t7_body, grid=(52,),
    in_specs=[pl.BlockSpec((1024, 32), lambda i: (i, 0)),
              pl.BlockSpec((1024, 32), lambda i: (i, 0)),
              pl.BlockSpec((1, 32), lambda i: (0, 0)),
              pl.BlockSpec((1, 1), lambda i: (0, 0))],
    out_specs=pl.BlockSpec((1024, 1), lambda i: (i, 0)),
    out_shape=_sds((_F2, 1)))


def _pad_spread(a, n, base, mod):
    fill = base + (jnp.arange(n - a.shape[0], dtype=I32) % mod)
    return jnp.concatenate([a.astype(I32), fill])


def kernel(x, edge1, pos, idx, ei2, emb, W1, b1, W2a, b2a, W2b, b2b, Wp, bp):
    s1r = _pad_spread(edge1[0], _E1PAD, _N, _N2 - _N).reshape(_E1R, 128)
    d1r = _pad_spread(edge1[1], _E1PAD, _N, _N2 - _N).reshape(_E1R, 128)
    s2r = _pad_spread(ei2[0], _E2PAD, _P, _P2 - _P).reshape(_E2R, 128)
    d2r = _pad_spread(ei2[1], _E2PAD, _P, _P2 - _P).reshape(_E2R, 128)
    xr = _pad_spread(x, _N2, 0, _N).reshape(_NR, 128)
    pos0r = _pad_spread(pos[:, 0], _P2, 0, _N).reshape(_PR, 128)
    pos1r = _pad_spread(pos[:, 1], _P2, 0, _N).reshape(_PR, 128)
    idx2 = idx.astype(I32).reshape(_F, 2)
    ier = _pad_spread(idx2[:, 0], _F2, 0, _P).reshape(_FR, 128)
    ior = _pad_spread(idx2[:, 1], _F2, 0, _P).reshape(_FR, 128)

    deg1p, degap, degbp, g0 = _s1_call(d1r, s2r, d2r, xr, emb)
    deg1c = (deg1p[0] + deg1p[1] + 1.0).reshape(_N2, 1)
    degac = (degap[0] + degap[1] + 1.0).reshape(_P2, 1)
    degbc = (degbp[0] + degbp[1] + 1.0).reshape(_P2, 1)

    hw1p = _t1_call(g0, W1, deg1c)
    accp = _conv1_call(s1r, d1r, hw1p)
    h1 = _t4_call(accp, hw1p, deg1c, b1.reshape(1, 32))

    ga, gb = _pair_call(pos0r, pos1r, h1)
    hap2, hbp2 = _t5_call(ga, gb, W2a, W2b, degac, degbc)
    acca2, accb2 = _conv2_call(s2r, d2r, hap2, hbp2)
    h2 = _t6_call(acca2, accb2, degac, degbc, b2a.reshape(1, 32),
                  b2b.reshape(1, 32))

    he, ho = _final_call(ier, ior, h2)
    out = _t7_call(he, ho, Wp.reshape(1, 32), bp.reshape(1, 1))
    return out[:_F]


# speedup vs baseline: 66.8525x; 1.4178x over previous
import jax
import jax.numpy as jnp
from jax import lax
from jax.experimental import pallas as pl
from jax.experimental.pallas import tpu as pltpu
from jax.experimental.pallas import tpu_sc as plsc

F32 = jnp.float32
I32 = jnp.int32
HI = lax.Precision.HIGHEST

_N = 10000
_N2 = 10240
_NR = _N2 // 128
_E1PAD = 655360
_E1R = _E1PAD // 128
_P = 100000
_P2 = 102400
_PR = _P2 // 128
_E2PAD = 1605632
_E2R = _E2PAD // 128
_F = 50000
_F2 = 53248
_FR = _F2 // 128
_H = 16

_NC, _NS = 2, 16
_MESH = plsc.VectorSubcoreMesh(
    core_axis_name="c", subcore_axis_name="s", num_cores=_NC, num_subcores=_NS)
_SC_PARAMS = pltpu.CompilerParams(use_tc_tiling_on_sc=False)


def _rs(d):
    return lax.rsqrt(jnp.maximum(d, 1e-12))


def _count_gather_body(d1r, s2r, d2r, xr, emb,
                       deg1p, degap, degbp, g0,
                       deg1_sh, dega_sh, degb_sh,
                       ones_v, zero_v, idx_v, gidx_v, grow_v, sem, semx):
    cid = lax.axis_index("c")
    sid = lax.axis_index("s")

    for i in range(8):
        ones_v[pl.ds(i * 16, 16)] = jnp.ones((16,), F32)

    def _z(i, c):
        zero_v[pl.ds(i * 16, 16)] = jnp.zeros((16,), F32)
        return c
    lax.fori_loop(0, 400, _z, 0)

    pltpu.sync_copy(zero_v.at[pl.ds(0, 640)], deg1_sh.at[pl.ds(sid * 640, 640)])
    pltpu.sync_copy(zero_v, dega_sh.at[pl.ds(sid * 6400, 6400)])
    pltpu.sync_copy(zero_v, degb_sh.at[pl.ds(sid * 6400, 6400)])
    plsc.subcore_barrier()

    def _count(idx_hbm, acc_sh, nrows_tile):
        base = cid * (_NS * nrows_tile) + sid * nrows_tile
        ng = nrows_tile // 8

        def _fire(parity, g):
            for j in range(8):
                pltpu.async_copy(ones_v, acc_sh.at[idx_v.at[parity].at[j]], sem,
                                 add=True)

        def _drain(parity):
            for j in range(8):
                pltpu.make_async_copy(ones_v, acc_sh.at[idx_v.at[parity].at[j]],
                                      sem).wait()

        pltpu.sync_copy(idx_hbm.at[pl.ds(base, 8)], idx_v.at[0])
        _fire(0, 0)

        def body(g, c):
            p = lax.rem(g, 2)
            q = 1 - p
            pltpu.sync_copy(idx_hbm.at[pl.ds(base + (g + 1) * 8, 8)], idx_v.at[q])
            _drain(p)
            _fire(q, g + 1)
            return c
        lax.fori_loop(0, ng - 1, body, 0)
        _drain(lax.rem(ng - 1, 2))

    _count(d1r, deg1_sh, 160)
    _count(d2r, dega_sh, 392)
    _count(s2r, degb_sh, 392)

    wid = sid * _NC + cid

    @pl.when(wid < 20)
    def _():
        base = wid * 4
        pltpu.sync_copy(xr.at[pl.ds(base, 4)], gidx_v)
        descs = [pltpu.async_copy(emb.at[gidx_v.at[j]], grow_v.at[j], semx)
                 for j in range(4)]
        for d in descs:
            d.wait()
        descs = [pltpu.async_copy(grow_v.at[j], g0.at[pl.ds((base + j) * 128, 128)],
                                  semx) for j in range(4)]
        for d in descs:
            d.wait()

    plsc.subcore_barrier()
    pltpu.sync_copy(deg1_sh.at[pl.ds(sid * 640, 640)],
                    deg1p.at[cid].at[pl.ds(sid * 640, 640)])
    pltpu.sync_copy(dega_sh.at[pl.ds(sid * 6400, 6400)],
                    degap.at[cid].at[pl.ds(sid * 6400, 6400)])
    pltpu.sync_copy(degb_sh.at[pl.ds(sid * 6400, 6400)],
                    degbp.at[cid].at[pl.ds(sid * 6400, 6400)])


def _edge_pipeline(gir, sir, tbl, acc_sh, gi_v, si_v, rows_v, semg, sems,
                   semi, base, ng, k):
    def _fire_g(rp, ip):
        for j in range(k):
            pltpu.async_copy(tbl.at[gi_v.at[ip].at[j]], rows_v.at[rp].at[j], semg)

    def _wait_g(rp, ip):
        for j in range(k):
            pltpu.make_async_copy(tbl.at[gi_v.at[ip].at[j]], rows_v.at[rp].at[j],
                                  semg).wait()

    def _fire_s(rp, ip):
        for j in range(k):
            pltpu.async_copy(rows_v.at[rp].at[j], acc_sh.at[si_v.at[ip].at[j]],
                             sems, add=True)

    def _wait_s(rp, ip):
        for j in range(k):
            pltpu.make_async_copy(rows_v.at[rp].at[j], acc_sh.at[si_v.at[ip].at[j]],
                                  sems).wait()

    def _fire_i(g, slot):
        pltpu.async_copy(gir.at[pl.ds(base + g * k, k)], gi_v.at[slot], semi)
        pltpu.async_copy(sir.at[pl.ds(base + g * k, k)], si_v.at[slot], semi)

    def _wait_i(g, slot):
        pltpu.make_async_copy(gir.at[pl.ds(base + g * k, k)], gi_v.at[slot],
                              semi).wait()
        pltpu.make_async_copy(sir.at[pl.ds(base + g * k, k)], si_v.at[slot],
                              semi).wait()

    pltpu.sync_copy(gir.at[pl.ds(base, k)], gi_v.at[0])
    pltpu.sync_copy(sir.at[pl.ds(base, k)], si_v.at[0])
    _fire_g(0, 0)
    _fire_i(1, 1)

    def body(g, c):
        rp = lax.rem(g, 2)
        rq = 1 - rp
        ip = lax.rem(g, 3)
        iq = lax.rem(g + 1, 3)
        ir = lax.rem(g + 2, 3)
        _wait_i(g + 1, iq)
        _wait_g(rp, ip)
        _fire_g(rq, iq)

        @pl.when(g + 2 <= ng - 1)
        def _():
            _fire_i(g + 2, ir)
        _fire_s(rp, ip)
        _wait_s(rp, ip)
        return c
    lax.fori_loop(0, ng - 1, body, 0)
    last = ng - 1
    _wait_g(lax.rem(last, 2), lax.rem(last, 3))
    _fire_s(lax.rem(last, 2), lax.rem(last, 3))
    _wait_s(lax.rem(last, 2), lax.rem(last, 3))


def _conv1_body(s1r, d1r, hw1p, accp,
                acc_sh, si_v, di_v, rows_v, semg, sems, semi):
    cid = lax.axis_index("c")
    sid = lax.axis_index("s")
    pltpu.sync_copy(hw1p.at[pl.ds(sid * 640, 640)], acc_sh.at[pl.ds(sid * 640, 640)])
    plsc.subcore_barrier()
    base = cid * (_NS * 160) + sid * 160
    _edge_pipeline(s1r, d1r, hw1p, acc_sh, si_v, di_v, rows_v, semg, sems,
                   semi, base, 20, 8)
    plsc.subcore_barrier()
    pltpu.sync_copy(acc_sh.at[pl.ds(sid * 640, 640)],
                    accp.at[cid].at[pl.ds(sid * 640, 640)])


def _conv2_body(s2r, d2r, tbla, tblb, acca, accb,
                acc_sh, gi_v, si_v, rows_v, semg, sems, semi):
    cid = lax.axis_index("c")
    sid = lax.axis_index("s")
    base = sid * 784
    for gir, sir, tbl, out in ((s2r, d2r, tbla, acca), (d2r, s2r, tblb, accb)):
        pltpu.sync_copy(tbl.at[cid].at[pl.ds(sid * 6400, 6400)],
                        acc_sh.at[pl.ds(sid * 6400, 6400)])
        plsc.subcore_barrier()
        _edge_pipeline(gir, sir, tbl.at[cid], acc_sh, gi_v, si_v, rows_v,
                       semg, sems, semi, base, 196, 4)
        plsc.subcore_barrier()
        pltpu.sync_copy(acc_sh.at[pl.ds(sid * 6400, 6400)],
                        out.at[cid].at[pl.ds(sid * 6400, 6400)])
        plsc.subcore_barrier()


def _gather_pipeline(idxr, tbl, outr, idx_v, rows_v, semg, semw, base, ng, k):
    def _fire_g(p, off):
        for j in range(k):
            pltpu.async_copy(tbl.at[idx_v.at[p].at[j]], rows_v.at[p].at[j], semg)

    def _wait_g(p):
        for j in range(k):
            pltpu.make_async_copy(tbl.at[idx_v.at[p].at[j]], rows_v.at[p].at[j],
                                  semg).wait()

    def _fire_w(p, grp):
        for j in range(k):
            pltpu.async_copy(rows_v.at[p].at[j],
                             outr.at[pl.ds((base + grp * k + j) * 128, 128)],
                             semw)

    def _wait_w(p, grp):
        for j in range(k):
            pltpu.make_async_copy(rows_v.at[p].at[j],
                                  outr.at[pl.ds((base + grp * k + j) * 128, 128)],
                                  semw).wait()

    pltpu.sync_copy(idxr.at[pl.ds(base, k)], idx_v.at[0])
    _fire_g(0, 0)

    def body(g, c):
        p = lax.rem(g, 2)
        q = 1 - p
        pltpu.sync_copy(idxr.at[pl.ds(base + (g + 1) * k, k)], idx_v.at[q])
        _wait_g(p)
        _fire_g(q, g + 1)
        _fire_w(p, g)
        _wait_w(p, g)
        return c
    lax.fori_loop(0, ng - 1, body, 0)
    last = lax.rem(ng - 1, 2)
    _wait_g(last)
    _fire_w(last, ng - 1)
    _wait_w(last, ng - 1)


def _pair_body(pos0r, pos1r, h1, ga, gb, idx_v, rows_v, semg, semw):
    wid = lax.axis_index("s") * _NC + lax.axis_index("c")
    base = wid * 25
    _gather_pipeline(pos0r, h1, ga, idx_v, rows_v, semg, semw, base, 5, 5)
    _gather_pipeline(pos1r, h1, gb, idx_v, rows_v, semg, semw, base, 5, 5)


def _final_body(ier, ior, h2l, h2h, hel, heh, hol, hoh, idx_v, rows_v,
                semg, semw):
    wid = lax.axis_index("s") * _NC + lax.axis_index("c")
    base = wid * 13
    for idxr, tbl, outr in ((ier, h2l, hel), (ier, h2h, heh),
                            (ior, h2l, hol), (ior, h2h, hoh)):
        _gather_pipeline(idxr, tbl, outr, idx_v, rows_v, semg, semw, base, 2, 6)
        pltpu.sync_copy(idxr.at[pl.ds(base + 12, 1)], idx_v.at[0].at[pl.ds(0, 1)])
        pltpu.async_copy(tbl.at[idx_v.at[0].at[0]], rows_v.at[0].at[0], semg).wait()
        pltpu.async_copy(rows_v.at[0].at[0],
                         outr.at[pl.ds((base + 12) * 128, 128)], semw).wait()


def _t1_body(g0_ref, w_ref, deg_ref, o_ref):
    o_ref[...] = jnp.dot(g0_ref[...], w_ref[...], precision=HI) * _rs(deg_ref[...])


def _t4_body(accp_ref, hw_ref, deg_ref, b_ref, o_ref):
    acc = accp_ref[0] + accp_ref[1] - hw_ref[...]
    o_ref[...] = jax.nn.relu(_rs(deg_ref[...]) * acc + b_ref[...])


def _t5_body(ga_ref, gb_ref, wa_ref, wb_ref, dega_ref, degb_ref, oa_ref, ob_ref):
    hp = ga_ref[...] * gb_ref[...]
    ha = jnp.dot(hp * _rs(dega_ref[...]), wa_ref[...], precision=HI)
    hb = jnp.dot(hp * _rs(degb_ref[...]), wb_ref[...], precision=HI)
    oa_ref[0] = ha[:, :64]
    oa_ref[1] = ha[:, 64:]
    ob_ref[0] = hb[:, :64]
    ob_ref[1] = hb[:, 64:]


def _t6_body(acca_ref, accb_ref, dega_ref, degb_ref, bal_ref, bah_ref,
             bbl_ref, bbh_ref, ol_ref, oh_ref):
    rsa = _rs(dega_ref[...])
    rsb = _rs(degb_ref[...])
    ol_ref[...] = (jax.nn.relu(rsa * acca_ref[0] + bal_ref[...])
                   + jax.nn.relu(rsb * accb_ref[0] + bbl_ref[...]))
    oh_ref[...] = (jax.nn.relu(rsa * acca_ref[1] + bah_ref[...])
                   + jax.nn.relu(rsb * accb_ref[1] + bbh_ref[...]))


def _t7_body(hel_ref, hol_ref, heh_ref, hoh_ref, wl_ref, wh_ref, s_ref, b_ref,
             o_ref):
    prod = jnp.concatenate([hel_ref[...] * hol_ref[...] * wl_ref[...],
                            heh_ref[...] * hoh_ref[...] * wh_ref[...]], axis=1)
    o_ref[...] = jnp.dot(prod, s_ref[...], precision=HI) + b_ref[...]


def _sds(shape, dtype=F32):
    return jax.ShapeDtypeStruct(shape, dtype)


_s1_call = pl.kernel(
    _count_gather_body,
    out_type=(_sds((2, _N2)), _sds((2, _P2)), _sds((2, _P2)), _sds((_N2, 128))),
    mesh=_MESH,
    compiler_params=_SC_PARAMS,
    scratch_types=[
        pltpu.VMEM_SHARED((_N2,), F32),
        pltpu.VMEM_SHARED((_P2,), F32),
        pltpu.VMEM_SHARED((_P2,), F32),
        pltpu.VMEM((128,), F32),
        pltpu.VMEM((6400,), F32),
        pltpu.VMEM((2, 8, 128), I32),
        pltpu.VMEM((4, 128), I32),
        pltpu.VMEM((4, 128, 128), F32),
        pltpu.SemaphoreType.DMA,
        pltpu.SemaphoreType.DMA,
    ])

_conv1_call = pl.kernel(
    _conv1_body,
    out_type=_sds((2, _N2, 32)),
    mesh=_MESH,
    compiler_params=_SC_PARAMS,
    scratch_types=[
        pltpu.VMEM_SHARED((_N2, 32), F32),
        pltpu.VMEM((3, 8, 128), I32),
        pltpu.VMEM((3, 8, 128), I32),
        pltpu.VMEM((2, 8, 128, 32), F32),
        pltpu.SemaphoreType.DMA,
        pltpu.SemaphoreType.DMA,
        pltpu.SemaphoreType.DMA,
    ])

_conv2_call = pl.kernel(
    _conv2_body,
    out_type=(_sds((2, _P2, _H)), _sds((2, _P2, _H))),
    mesh=_MESH,
    compiler_params=_SC_PARAMS,
    scratch_types=[
        pltpu.VMEM_SHARED((_P2, _H), F32),
        pltpu.VMEM((3, 4, 128), I32),
        pltpu.VMEM((3, 4, 128), I32),
        pltpu.VMEM((2, 4, 128, _H), F32),
        pltpu.SemaphoreType.DMA,
        pltpu.SemaphoreType.DMA,
        pltpu.SemaphoreType.DMA,
    ])

_pair_call = pl.kernel(
    _pair_body,
    out_type=(_sds((_P2, 32)), _sds((_P2, 32))),
    mesh=_MESH,
    compiler_params=_SC_PARAMS,
    scratch_types=[
        pltpu.VMEM((2, 5, 128), I32),
        pltpu.VMEM((2, 5, 128, 32), F32),
        pltpu.SemaphoreType.DMA,
        pltpu.SemaphoreType.DMA,
    ])

_final_call = pl.kernel(
    _final_body,
    out_type=(_sds((_F2, _H)), _sds((_F2, _H)), _sds((_F2, _H)),
              _sds((_F2, _H))),
    mesh=_MESH,
    compiler_params=_SC_PARAMS,
    scratch_types=[
        pltpu.VMEM((2, 6, 128), I32),
        pltpu.VMEM((2, 6, 128, _H), F32),
        pltpu.SemaphoreType.DMA,
        pltpu.SemaphoreType.DMA,
    ])

_t1_call = pl.pallas_call(
    _t1_body, grid=(10,),
    in_specs=[pl.BlockSpec((1024, 128), lambda i: (i, 0)),
              pl.BlockSpec((128, 32), lambda i: (0, 0)),
              pl.BlockSpec((1024, 1), lambda i: (i, 0))],
    out_specs=pl.BlockSpec((1024, 32), lambda i: (i, 0)),
    out_shape=_sds((_N2, 32)))

_t4_call = pl.pallas_call(
    _t4_body, grid=(10,),
    in_specs=[pl.BlockSpec((2, 1024, 32), lambda i: (0, i, 0)),
              pl.BlockSpec((1024, 32), lambda i: (i, 0)),
              pl.BlockSpec((1024, 1), lambda i: (i, 0)),
              pl.BlockSpec((1, 32), lambda i: (0, 0))],
    out_specs=pl.BlockSpec((1024, 32), lambda i: (i, 0)),
    out_shape=_sds((_N2, 32)))

_t5_call = pl.pallas_call(
    _t5_body, grid=(25,),
    in_specs=[pl.BlockSpec((1024, 128), lambda i: (i, 0)),
              pl.BlockSpec((1024, 128), lambda i: (i, 0)),
              pl.BlockSpec((128, 128), lambda i: (0, 0)),
              pl.BlockSpec((128, 128), lambda i: (0, 0)),
              pl.BlockSpec((1024, 128), lambda i: (i, 0)),
              pl.BlockSpec((1024, 128), lambda i: (i, 0))],
    out_specs=(pl.BlockSpec((2, 1024, 64), lambda i: (0, i, 0)),
               pl.BlockSpec((2, 1024, 64), lambda i: (0, i, 0))),
    out_shape=(_sds((2, _P2 // 4, 64)), _sds((2, _P2 // 4, 64))))

_t6_call = pl.pallas_call(
    _t6_body, grid=(25,),
    in_specs=[pl.BlockSpec((2, 1024, 64), lambda i: (0, i, 0)),
              pl.BlockSpec((2, 1024, 64), lambda i: (0, i, 0)),
              pl.BlockSpec((1024, 64), lambda i: (i, 0)),
              pl.BlockSpec((1024, 64), lambda i: (i, 0)),
              pl.BlockSpec((1, 64), lambda i: (0, 0)),
              pl.BlockSpec((1, 64), lambda i: (0, 0)),
              pl.BlockSpec((1, 64), lambda i: (0, 0)),
              pl.BlockSpec((1, 64), lambda i: (0, 0))],
    out_specs=(pl.BlockSpec((1024, 64), lambda i: (i, 0)),
               pl.BlockSpec((1024, 64), lambda i: (i, 0))),
    out_shape=(_sds((_P2 // 4, 64)), _sds((_P2 // 4, 64))))

_t7_call = pl.pallas_call(
    _t7_body, grid=(13,),
    in_specs=[pl.BlockSpec((1024, 64), lambda i: (i, 0)),
              pl.BlockSpec((1024, 64), lambda i: (i, 0)),
              pl.BlockSpec((1024, 64), lambda i: (i, 0)),
              pl.BlockSpec((1024, 64), lambda i: (i, 0)),
              pl.BlockSpec((1, 64), lambda i: (0, 0)),
              pl.BlockSpec((1, 64), lambda i: (0, 0)),
              pl.BlockSpec((128, 4), lambda i: (0, 0)),
              pl.BlockSpec((1, 1), lambda i: (0, 0))],
    out_specs=pl.BlockSpec((1024, 4), lambda i: (i, 0)),
    out_shape=_sds((_F2 // 4, 4)))


def _pad_spread(a, n, base, mod):
    fill = base + (jnp.arange(n - a.shape[0], dtype=I32) % mod)
    return jnp.concatenate([a.astype(I32), fill])


def kernel(x, edge1, pos, idx, ei2, emb, W1, b1, W2a, b2a, W2b, b2b, Wp, bp):
    s1r = _pad_spread(edge1[0], _E1PAD, _N, _N2 - _N).reshape(_E1R, 128)
    d1r = _pad_spread(edge1[1], _E1PAD, _N, _N2 - _N).reshape(_E1R, 128)
    s2r = _pad_spread(ei2[0], _E2PAD, _P, _P2 - _P).reshape(_E2R, 128)
    d2r = _pad_spread(ei2[1], _E2PAD, _P, _P2 - _P).reshape(_E2R, 128)
    xr = _pad_spread(x, _N2, 0, _N).reshape(_NR, 128)
    pos0r = _pad_spread(pos[:, 0], _P2, 0, _N).reshape(_PR, 128)
    pos1r = _pad_spread(pos[:, 1], _P2, 0, _N).reshape(_PR, 128)
    idx2 = idx.astype(I32).reshape(_F, 2)
    ier = _pad_spread(idx2[:, 0], _F2, 0, _P).reshape(_FR, 128)
    ior = _pad_spread(idx2[:, 1], _F2, 0, _P).reshape(_FR, 128)

    deg1p, degap, degbp, g0 = _s1_call(d1r, s2r, d2r, xr, emb)
    deg1c = (deg1p[0] + deg1p[1] + 1.0).reshape(_N2, 1)
    degac = (degap[0] + degap[1] + 1.0).reshape(_P2, 1)
    degbc = (degbp[0] + degbp[1] + 1.0).reshape(_P2, 1)

    hw1p = _t1_call(g0, W1, deg1c)
    accp = _conv1_call(s1r, d1r, hw1p)
    h1 = _t4_call(accp, hw1p, deg1c, b1.reshape(1, 32))

    ga, gb = _pair_call(pos0r, pos1r, h1)

    dega4 = jnp.broadcast_to(degac, (_P2, 32)).reshape(_P2 // 4, 128)
    degb4 = jnp.broadcast_to(degbc, (_P2, 32)).reshape(_P2 // 4, 128)
    dega2 = jnp.broadcast_to(degac, (_P2, _H)).reshape(_P2 // 4, 64)
    degb2 = jnp.broadcast_to(degbc, (_P2, _H)).reshape(_P2 // 4, 64)
    eye4 = jnp.eye(4, dtype=F32)
    wacat = jnp.concatenate([jnp.kron(eye4, W2a[:, :_H]),
                             jnp.kron(eye4, W2a[:, _H:])], axis=1)
    wbcat = jnp.concatenate([jnp.kron(eye4, W2b[:, :_H]),
                             jnp.kron(eye4, W2b[:, _H:])], axis=1)

    hap2, hbp2 = _t5_call(ga.reshape(_P2 // 4, 128), gb.reshape(_P2 // 4, 128),
                          wacat, wbcat, dega4, degb4)
    acca2, accb2 = _conv2_call(s2r, d2r, hap2.reshape(2, _P2, _H),
                               hbp2.reshape(2, _P2, _H))
    h2l, h2h = _t6_call(acca2.reshape(2, _P2 // 4, 64),
                        accb2.reshape(2, _P2 // 4, 64), dega2, degb2,
                        jnp.tile(b2a[:_H], 4).reshape(1, 64),
                        jnp.tile(b2a[_H:], 4).reshape(1, 64),
                        jnp.tile(b2b[:_H], 4).reshape(1, 64),
                        jnp.tile(b2b[_H:], 4).reshape(1, 64))

    hel, heh, hol, hoh = _final_call(ier, ior, h2l.reshape(_P2, _H),
                                     h2h.reshape(_P2, _H))
    sel = jax.nn.one_hot((jnp.arange(128) % 64) // _H, 4, dtype=F32)
    out4 = _t7_call(hel.reshape(_F2 // 4, 64), hol.reshape(_F2 // 4, 64),
                    heh.reshape(_F2 // 4, 64), hoh.reshape(_F2 // 4, 64),
                    jnp.tile(Wp[:_H, 0], 4).reshape(1, 64),
                    jnp.tile(Wp[_H:, 0], 4).reshape(1, 64),
                    sel, bp.reshape(1, 1))
    return out4.reshape(_F2, 1)[:_F]


# speedup vs baseline: 71.0392x; 1.0626x over previous
import jax
import jax.numpy as jnp
from jax import lax
from jax.experimental import pallas as pl
from jax.experimental.pallas import tpu as pltpu
from jax.experimental.pallas import tpu_sc as plsc

F32 = jnp.float32
I32 = jnp.int32
HI = lax.Precision.HIGHEST

_N = 10000
_N2 = 10240
_NR = _N2 // 128
_E1PAD = 655360
_E1R = _E1PAD // 128
_P = 100000
_P2 = 102400
_PR = _P2 // 128
_E2PAD = 1605632
_E2R = _E2PAD // 128
_F = 50000
_F2 = 53248
_FR = _F2 // 128
_H = 16

_NC, _NS = 2, 16
_MESH = plsc.VectorSubcoreMesh(
    core_axis_name="c", subcore_axis_name="s", num_cores=_NC, num_subcores=_NS)
_SC_PARAMS = pltpu.CompilerParams(use_tc_tiling_on_sc=False)


def _rs(d):
    return lax.rsqrt(jnp.maximum(d, 1e-12))


def _zero_fill(zero_v, n):
    def _z(i, c):
        zero_v[pl.ds(i * 16, 16)] = jnp.zeros((16,), F32)
        return c
    lax.fori_loop(0, n // 16, _z, 0)


def _ones_fill(ones_v):
    for i in range(8):
        ones_v[pl.ds(i * 16, 16)] = jnp.ones((16,), F32)


def _count_stream(idx_hbm, acc_sh, ones_v, idx_v, sem, base, ng):
    def _fire(parity):
        for j in range(8):
            pltpu.async_copy(ones_v, acc_sh.at[idx_v.at[parity].at[j]], sem,
                             add=True)

    def _drain(parity):
        for j in range(8):
            pltpu.make_async_copy(ones_v, acc_sh.at[idx_v.at[parity].at[j]],
                                  sem).wait()

    pltpu.sync_copy(idx_hbm.at[pl.ds(base, 8)], idx_v.at[0])
    _fire(0)

    def body(g, c):
        p = lax.rem(g, 2)
        q = 1 - p
        pltpu.sync_copy(idx_hbm.at[pl.ds(base + (g + 1) * 8, 8)], idx_v.at[q])
        _drain(p)
        _fire(q)
        return c
    lax.fori_loop(0, ng - 1, body, 0)
    _drain(lax.rem(ng - 1, 2))


def _s1a_body(d1r, xr, emb, deg1p, g0,
              deg1_sh, ones_v, zero_v, idx_v, gidx_v, grow_v, sem, semx):
    cid = lax.axis_index("c")
    sid = lax.axis_index("s")
    _ones_fill(ones_v)
    _zero_fill(zero_v, 640)
    pltpu.sync_copy(zero_v, deg1_sh.at[pl.ds(sid * 640, 640)])
    plsc.subcore_barrier()
    _count_stream(d1r, deg1_sh, ones_v, idx_v, sem,
                  cid * (_NS * 160) + sid * 160, 20)

    wid = sid * _NC + cid

    @pl.when(wid < 20)
    def _():
        base = wid * 4
        pltpu.sync_copy(xr.at[pl.ds(base, 4)], gidx_v)
        descs = [pltpu.async_copy(emb.at[gidx_v.at[j]], grow_v.at[j], semx)
                 for j in range(4)]
        for d in descs:
            d.wait()
        descs = [pltpu.async_copy(grow_v.at[j], g0.at[pl.ds((base + j) * 128, 128)],
                                  semx) for j in range(4)]
        for d in descs:
            d.wait()

    plsc.subcore_barrier()
    pltpu.sync_copy(deg1_sh.at[pl.ds(sid * 640, 640)],
                    deg1p.at[cid].at[pl.ds(sid * 640, 640)])


def _s1b_body(s2r, d2r, degap, degbp,
              dega_sh, degb_sh, ones_v, zero_v, idx_v, sem):
    cid = lax.axis_index("c")
    sid = lax.axis_index("s")
    _ones_fill(ones_v)
    _zero_fill(zero_v, 6400)
    pltpu.sync_copy(zero_v, dega_sh.at[pl.ds(sid * 6400, 6400)])
    pltpu.sync_copy(zero_v, degb_sh.at[pl.ds(sid * 6400, 6400)])
    plsc.subcore_barrier()
    base = cid * (_NS * 392) + sid * 392
    _count_stream(d2r, dega_sh, ones_v, idx_v, sem, base, 49)
    _count_stream(s2r, degb_sh, ones_v, idx_v, sem, base, 49)
    plsc.subcore_barrier()
    pltpu.sync_copy(dega_sh.at[pl.ds(sid * 6400, 6400)],
                    degap.at[cid].at[pl.ds(sid * 6400, 6400)])
    pltpu.sync_copy(degb_sh.at[pl.ds(sid * 6400, 6400)],
                    degbp.at[cid].at[pl.ds(sid * 6400, 6400)])


def _edge_pipeline(gir, sir, tbl, acc_sh, gi_v, si_v, rows_v, semg, sems,
                   semi, base, ng, k):
    def _fire_g(rp, ip):
        for j in range(k):
            pltpu.async_copy(tbl.at[gi_v.at[ip].at[j]], rows_v.at[rp].at[j], semg)

    def _wait_g(rp, ip):
        for j in range(k):
            pltpu.make_async_copy(tbl.at[gi_v.at[ip].at[j]], rows_v.at[rp].at[j],
                                  semg).wait()

    def _fire_s(rp, ip):
        for j in range(k):
            pltpu.async_copy(rows_v.at[rp].at[j], acc_sh.at[si_v.at[ip].at[j]],
                             sems, add=True)

    def _wait_s(rp, ip):
        for j in range(k):
            pltpu.make_async_copy(rows_v.at[rp].at[j], acc_sh.at[si_v.at[ip].at[j]],
                                  sems).wait()

    def _fire_i(g, slot):
        pltpu.async_copy(gir.at[pl.ds(base + g * k, k)], gi_v.at[slot], semi)
        pltpu.async_copy(sir.at[pl.ds(base + g * k, k)], si_v.at[slot], semi)

    def _wait_i(g, slot):
        pltpu.make_async_copy(gir.at[pl.ds(base + g * k, k)], gi_v.at[slot],
                              semi).wait()
        pltpu.make_async_copy(sir.at[pl.ds(base + g * k, k)], si_v.at[slot],
                              semi).wait()

    pltpu.sync_copy(gir.at[pl.ds(base, k)], gi_v.at[0])
    pltpu.sync_copy(sir.at[pl.ds(base, k)], si_v.at[0])
    _fire_g(0, 0)
    _fire_i(1, 1)

    def body(g, c):
        rp = lax.rem(g, 2)
        rq = 1 - rp
        ip = lax.rem(g, 3)
        iq = lax.rem(g + 1, 3)
        ir = lax.rem(g + 2, 3)
        _wait_i(g + 1, iq)
        _wait_g(rp, ip)
        _fire_g(rq, iq)

        @pl.when(g + 2 <= ng - 1)
        def _():
            _fire_i(g + 2, ir)
        _fire_s(rp, ip)
        _wait_s(rp, ip)
        return c
    lax.fori_loop(0, ng - 1, body, 0)
    last = ng - 1
    _wait_g(lax.rem(last, 2), lax.rem(last, 3))
    _fire_s(lax.rem(last, 2), lax.rem(last, 3))
    _wait_s(lax.rem(last, 2), lax.rem(last, 3))


def _conv1_body(s1r, d1r, hw1p, accp,
                acc_sh, si_v, di_v, rows_v, semg, sems, semi):
    cid = lax.axis_index("c")
    sid = lax.axis_index("s")
    pltpu.sync_copy(hw1p.at[pl.ds(sid * 640, 640)], acc_sh.at[pl.ds(sid * 640, 640)])
    plsc.subcore_barrier()
    base = cid * (_NS * 160) + sid * 160
    _edge_pipeline(s1r, d1r, hw1p, acc_sh, si_v, di_v, rows_v, semg, sems,
                   semi, base, 20, 8)
    plsc.subcore_barrier()
    pltpu.sync_copy(acc_sh.at[pl.ds(sid * 640, 640)],
                    accp.at[cid].at[pl.ds(sid * 640, 640)])


def _conv2_body(s2r, d2r, tbla, tblb, acca, accb,
                acc_sh, gi_v, si_v, rows_v, semg, sems, semi):
    cid = lax.axis_index("c")
    sid = lax.axis_index("s")
    base = sid * 784
    for gir, sir, tbl, out in ((s2r, d2r, tbla, acca), (d2r, s2r, tblb, accb)):
        pltpu.sync_copy(tbl.at[cid].at[pl.ds(sid * 6400, 6400)],
                        acc_sh.at[pl.ds(sid * 6400, 6400)])
        plsc.subcore_barrier()
        _edge_pipeline(gir, sir, tbl.at[cid], acc_sh, gi_v, si_v, rows_v,
                       semg, sems, semi, base, 196, 4)
        plsc.subcore_barrier()
        pltpu.sync_copy(acc_sh.at[pl.ds(sid * 6400, 6400)],
                        out.at[cid].at[pl.ds(sid * 6400, 6400)])
        plsc.subcore_barrier()


def _gather_pipeline(idxr, tbl, outr, idx_v, rows_v, semg, semw, base, ng, k):
    def _fire_g(p, off):
        for j in range(k):
            pltpu.async_copy(tbl.at[idx_v.at[p].at[j]], rows_v.at[p].at[j], semg)

    def _wait_g(p):
        for j in range(k):
            pltpu.make_async_copy(tbl.at[idx_v.at[p].at[j]], rows_v.at[p].at[j],
                                  semg).wait()

    def _fire_w(p, grp):
        for j in range(k):
            pltpu.async_copy(rows_v.at[p].at[j],
                             outr.at[pl.ds((base + grp * k + j) * 128, 128)],
                             semw)

    def _wait_w(p, grp):
        for j in range(k):
            pltpu.make_async_copy(rows_v.at[p].at[j],
                                  outr.at[pl.ds((base + grp * k + j) * 128, 128)],
                                  semw).wait()

    pltpu.sync_copy(idxr.at[pl.ds(base, k)], idx_v.at[0])
    _fire_g(0, 0)

    def body(g, c):
        p = lax.rem(g, 2)
        q = 1 - p
        pltpu.sync_copy(idxr.at[pl.ds(base + (g + 1) * k, k)], idx_v.at[q])
        _wait_g(p)
        _fire_g(q, g + 1)
        _fire_w(p, g)
        _wait_w(p, g)
        return c
    lax.fori_loop(0, ng - 1, body, 0)
    last = lax.rem(ng - 1, 2)
    _wait_g(last)
    _fire_w(last, ng - 1)
    _wait_w(last, ng - 1)


def _pair_body(pos0r, pos1r, h1, ga, gb, idx_v, rows_v, semg, semw):
    wid = lax.axis_index("s") * _NC + lax.axis_index("c")
    base = wid * 25
    _gather_pipeline(pos0r, h1, ga, idx_v, rows_v, semg, semw, base, 5, 5)
    _gather_pipeline(pos1r, h1, gb, idx_v, rows_v, semg, semw, base, 5, 5)


def _final_body(ier, ior, h2l, h2h, hel, heh, hol, hoh, idx_v, rows_v,
                semg, semw):
    wid = lax.axis_index("s") * _NC + lax.axis_index("c")
    base = wid * 13
    for idxr, tbl, outr in ((ier, h2l, hel), (ier, h2h, heh),
                            (ior, h2l, hol), (ior, h2h, hoh)):
        _gather_pipeline(idxr, tbl, outr, idx_v, rows_v, semg, semw, base, 2, 6)
        pltpu.sync_copy(idxr.at[pl.ds(base + 12, 1)], idx_v.at[0].at[pl.ds(0, 1)])
        pltpu.async_copy(tbl.at[idx_v.at[0].at[0]], rows_v.at[0].at[0], semg).wait()
        pltpu.async_copy(rows_v.at[0].at[0],
                         outr.at[pl.ds((base + 12) * 128, 128)], semw).wait()


def _t1_body(g0_ref, w_ref, deg_ref, o_ref):
    o_ref[...] = jnp.dot(g0_ref[...], w_ref[...], precision=HI) * _rs(deg_ref[...])


def _t4_body(accp_ref, hw_ref, deg_ref, b_ref, o_ref):
    acc = accp_ref[0] + accp_ref[1] - hw_ref[...]
    o_ref[...] = jax.nn.relu(_rs(deg_ref[...]) * acc + b_ref[...])


def _t5_body(ga_ref, gb_ref, wa_ref, wb_ref, dega_ref, degb_ref, oa_ref, ob_ref):
    hp = ga_ref[...] * gb_ref[...]
    ha = jnp.dot(hp * _rs(dega_ref[...]), wa_ref[...], precision=HI)
    hb = jnp.dot(hp * _rs(degb_ref[...]), wb_ref[...], precision=HI)
    oa_ref[0] = ha[:, :64]
    oa_ref[1] = ha[:, 64:]
    ob_ref[0] = hb[:, :64]
    ob_ref[1] = hb[:, 64:]


def _t6_body(acca_ref, accb_ref, dega_ref, degb_ref, bal_ref, bah_ref,
             bbl_ref, bbh_ref, ol_ref, oh_ref):
    rsa = _rs(dega_ref[...])
    rsb = _rs(degb_ref[...])
    ol_ref[...] = (jax.nn.relu(rsa * acca_ref[0] + bal_ref[...])
                   + jax.nn.relu(rsb * accb_ref[0] + bbl_ref[...]))
    oh_ref[...] = (jax.nn.relu(rsa * acca_ref[1] + bah_ref[...])
                   + jax.nn.relu(rsb * accb_ref[1] + bbh_ref[...]))


def _t7_body(hel_ref, hol_ref, heh_ref, hoh_ref, wl_ref, wh_ref, s_ref, b_ref,
             o_ref):
    prod = jnp.concatenate([hel_ref[...] * hol_ref[...] * wl_ref[...],
                            heh_ref[...] * hoh_ref[...] * wh_ref[...]], axis=1)
    o_ref[...] = jnp.dot(prod, s_ref[...], precision=HI) + b_ref[...]


def _sds(shape, dtype=F32):
    return jax.ShapeDtypeStruct(shape, dtype)


_s1a_call = pl.kernel(
    _s1a_body,
    out_type=(_sds((2, _N2)), _sds((_N2, 128))),
    mesh=_MESH,
    compiler_params=_SC_PARAMS,
    scratch_types=[
        pltpu.VMEM_SHARED((_N2,), F32),
        pltpu.VMEM((128,), F32),
        pltpu.VMEM((640,), F32),
        pltpu.VMEM((2, 8, 128), I32),
        pltpu.VMEM((4, 128), I32),
        pltpu.VMEM((4, 128, 128), F32),
        pltpu.SemaphoreType.DMA,
        pltpu.SemaphoreType.DMA,
    ])

_s1b_call = pl.kernel(
    _s1b_body,
    out_type=(_sds((2, _P2)), _sds((2, _P2))),
    mesh=_MESH,
    compiler_params=_SC_PARAMS,
    scratch_types=[
        pltpu.VMEM_SHARED((_P2,), F32),
        pltpu.VMEM_SHARED((_P2,), F32),
        pltpu.VMEM((128,), F32),
        pltpu.VMEM((6400,), F32),
        pltpu.VMEM((2, 8, 128), I32),
        pltpu.SemaphoreType.DMA,
    ])

_conv1_call = pl.kernel(
    _conv1_body,
    out_type=_sds((2, _N2, 32)),
    mesh=_MESH,
    compiler_params=_SC_PARAMS,
    scratch_types=[
        pltpu.VMEM_SHARED((_N2, 32), F32),
        pltpu.VMEM((3, 8, 128), I32),
        pltpu.VMEM((3, 8, 128), I32),
        pltpu.VMEM((2, 8, 128, 32), F32),
        pltpu.SemaphoreType.DMA,
        pltpu.SemaphoreType.DMA,
        pltpu.SemaphoreType.DMA,
    ])

_conv2_call = pl.kernel(
    _conv2_body,
    out_type=(_sds((2, _P2, _H)), _sds((2, _P2, _H))),
    mesh=_MESH,
    compiler_params=_SC_PARAMS,
    scratch_types=[
        pltpu.VMEM_SHARED((_P2, _H), F32),
        pltpu.VMEM((3, 4, 128), I32),
        pltpu.VMEM((3, 4, 128), I32),
        pltpu.VMEM((2, 4, 128, _H), F32),
        pltpu.SemaphoreType.DMA,
        pltpu.SemaphoreType.DMA,
        pltpu.SemaphoreType.DMA,
    ])

_pair_call = pl.kernel(
    _pair_body,
    out_type=(_sds((_P2, 32)), _sds((_P2, 32))),
    mesh=_MESH,
    compiler_params=_SC_PARAMS,
    scratch_types=[
        pltpu.VMEM((2, 5, 128), I32),
        pltpu.VMEM((2, 5, 128, 32), F32),
        pltpu.SemaphoreType.DMA,
        pltpu.SemaphoreType.DMA,
    ])

_final_call = pl.kernel(
    _final_body,
    out_type=(_sds((_F2, _H)), _sds((_F2, _H)), _sds((_F2, _H)),
              _sds((_F2, _H))),
    mesh=_MESH,
    compiler_params=_SC_PARAMS,
    scratch_types=[
        pltpu.VMEM((2, 6, 128), I32),
        pltpu.VMEM((2, 6, 128, _H), F32),
        pltpu.SemaphoreType.DMA,
        pltpu.SemaphoreType.DMA,
    ])

_t1_call = pl.pallas_call(
    _t1_body, grid=(10,),
    in_specs=[pl.BlockSpec((1024, 128), lambda i: (i, 0)),
              pl.BlockSpec((128, 32), lambda i: (0, 0)),
              pl.BlockSpec((1024, 1), lambda i: (i, 0))],
    out_specs=pl.BlockSpec((1024, 32), lambda i: (i, 0)),
    out_shape=_sds((_N2, 32)))

_t4_call = pl.pallas_call(
    _t4_body, grid=(10,),
    in_specs=[pl.BlockSpec((2, 1024, 32), lambda i: (0, i, 0)),
              pl.BlockSpec((1024, 32), lambda i: (i, 0)),
              pl.BlockSpec((1024, 1), lambda i: (i, 0)),
              pl.BlockSpec((1, 32), lambda i: (0, 0))],
    out_specs=pl.BlockSpec((1024, 32), lambda i: (i, 0)),
    out_shape=_sds((_N2, 32)))

_t5_call = pl.pallas_call(
    _t5_body, grid=(25,),
    in_specs=[pl.BlockSpec((1024, 128), lambda i: (i, 0)),
              pl.BlockSpec((1024, 128), lambda i: (i, 0)),
              pl.BlockSpec((128, 128), lambda i: (0, 0)),
              pl.BlockSpec((128, 128), lambda i: (0, 0)),
              pl.BlockSpec((1024, 128), lambda i: (i, 0)),
              pl.BlockSpec((1024, 128), lambda i: (i, 0))],
    out_specs=(pl.BlockSpec((2, 1024, 64), lambda i: (0, i, 0)),
               pl.BlockSpec((2, 1024, 64), lambda i: (0, i, 0))),
    out_shape=(_sds((2, _P2 // 4, 64)), _sds((2, _P2 // 4, 64))))

_t6_call = pl.pallas_call(
    _t6_body, grid=(25,),
    in_specs=[pl.BlockSpec((2, 1024, 64), lambda i: (0, i, 0)),
              pl.BlockSpec((2, 1024, 64), lambda i: (0, i, 0)),
              pl.BlockSpec((1024, 64), lambda i: (i, 0)),
              pl.BlockSpec((1024, 64), lambda i: (i, 0)),
              pl.BlockSpec((1, 64), lambda i: (0, 0)),
              pl.BlockSpec((1, 64), lambda i: (0, 0)),
              pl.BlockSpec((1, 64), lambda i: (0, 0)),
              pl.BlockSpec((1, 64), lambda i: (0, 0))],
    out_specs=(pl.BlockSpec((1024, 64), lambda i: (i, 0)),
               pl.BlockSpec((1024, 64), lambda i: (i, 0))),
    out_shape=(_sds((_P2 // 4, 64)), _sds((_P2 // 4, 64))))

_t7_call = pl.pallas_call(
    _t7_body, grid=(13,),
    in_specs=[pl.BlockSpec((1024, 64), lambda i: (i, 0)),
              pl.BlockSpec((1024, 64), lambda i: (i, 0)),
              pl.BlockSpec((1024, 64), lambda i: (i, 0)),
              pl.BlockSpec((1024, 64), lambda i: (i, 0)),
              pl.BlockSpec((1, 64), lambda i: (0, 0)),
              pl.BlockSpec((1, 64), lambda i: (0, 0)),
              pl.BlockSpec((128, 4), lambda i: (0, 0)),
              pl.BlockSpec((1, 1), lambda i: (0, 0))],
    out_specs=pl.BlockSpec((1024, 4), lambda i: (i, 0)),
    out_shape=_sds((_F2 // 4, 4)))


def _pad_spread(a, n, base, mod):
    fill = base + (jnp.arange(n - a.shape[0], dtype=I32) % mod)
    return jnp.concatenate([a.astype(I32), fill])


def kernel(x, edge1, pos, idx, ei2, emb, W1, b1, W2a, b2a, W2b, b2b, Wp, bp):
    s1r = _pad_spread(edge1[0], _E1PAD, _N, _N2 - _N).reshape(_E1R, 128)
    d1r = _pad_spread(edge1[1], _E1PAD, _N, _N2 - _N).reshape(_E1R, 128)
    s2r = _pad_spread(ei2[0], _E2PAD, _P, _P2 - _P).reshape(_E2R, 128)
    d2r = _pad_spread(ei2[1], _E2PAD, _P, _P2 - _P).reshape(_E2R, 128)
    xr = _pad_spread(x, _N2, 0, _N).reshape(_NR, 128)
    pos0r = _pad_spread(pos[:, 0], _P2, 0, _N).reshape(_PR, 128)
    pos1r = _pad_spread(pos[:, 1], _P2, 0, _N).reshape(_PR, 128)
    idx2 = idx.astype(I32).reshape(_F, 2)
    ier = _pad_spread(idx2[:, 0], _F2, 0, _P).reshape(_FR, 128)
    ior = _pad_spread(idx2[:, 1], _F2, 0, _P).reshape(_FR, 128)

    deg1p, g0 = _s1a_call(d1r, xr, emb)
    deg1c = (deg1p[0] + deg1p[1] + 1.0).reshape(_N2, 1)

    hw1p = _t1_call(g0, W1, deg1c)
    accp = _conv1_call(s1r, d1r, hw1p)
    h1 = _t4_call(accp, hw1p, deg1c, b1.reshape(1, 32))

    ga, gb = _pair_call(pos0r, pos1r, h1)

    degap, degbp = _s1b_call(s2r, d2r)
    degac = (degap[0] + degap[1] + 1.0).reshape(_P2, 1)
    degbc = (degbp[0] + degbp[1] + 1.0).reshape(_P2, 1)
    dega4 = jnp.broadcast_to(degac, (_P2, 32)).reshape(_P2 // 4, 128)
    degb4 = jnp.broadcast_to(degbc, (_P2, 32)).reshape(_P2 // 4, 128)
    dega2 = jnp.broadcast_to(degac, (_P2, _H)).reshape(_P2 // 4, 64)
    degb2 = jnp.broadcast_to(degbc, (_P2, _H)).reshape(_P2 // 4, 64)
    eye4 = jnp.eye(4, dtype=F32)
    wacat = jnp.concatenate([jnp.kron(eye4, W2a[:, :_H]),
                             jnp.kron(eye4, W2a[:, _H:])], axis=1)
    wbcat = jnp.concatenate([jnp.kron(eye4, W2b[:, :_H]),
                             jnp.kron(eye4, W2b[:, _H:])], axis=1)

    hap2, hbp2 = _t5_call(ga.reshape(_P2 // 4, 128), gb.reshape(_P2 // 4, 128),
                          wacat, wbcat, dega4, degb4)
    acca2, accb2 = _conv2_call(s2r, d2r, hap2.reshape(2, _P2, _H),
                               hbp2.reshape(2, _P2, _H))
    h2l, h2h = _t6_call(acca2.reshape(2, _P2 // 4, 64),
                        accb2.reshape(2, _P2 // 4, 64), dega2, degb2,
                        jnp.tile(b2a[:_H], 4).reshape(1, 64),
                        jnp.tile(b2a[_H:], 4).reshape(1, 64),
                        jnp.tile(b2b[:_H], 4).reshape(1, 64),
                        jnp.tile(b2b[_H:], 4).reshape(1, 64))

    hel, heh, hol, hoh = _final_call(ier, ior, h2l.reshape(_P2, _H),
                                     h2h.reshape(_P2, _H))
    sel = jax.nn.one_hot((jnp.arange(128) % 64) // _H, 4, dtype=F32)
    out4 = _t7_call(hel.reshape(_F2 // 4, 64), hol.reshape(_F2 // 4, 64),
                    heh.reshape(_F2 // 4, 64), hoh.reshape(_F2 // 4, 64),
                    jnp.tile(Wp[:_H, 0], 4).reshape(1, 64),
                    jnp.tile(Wp[_H:, 0], 4).reshape(1, 64),
                    sel, bp.reshape(1, 1))
    return out4.reshape(_F2, 1)[:_F]


# speedup vs baseline: 71.0414x; 1.0000x over previous
import jax
import jax.numpy as jnp
from jax import lax
from jax.experimental import pallas as pl
from jax.experimental.pallas import tpu as pltpu
from jax.experimental.pallas import tpu_sc as plsc

F32 = jnp.float32
I32 = jnp.int32
HI = lax.Precision.HIGHEST

_N = 10000
_N2 = 10240
_NR = _N2 // 128
_E1PAD = 655360
_E1R = _E1PAD // 128
_P = 100000
_P2 = 102400
_PR = _P2 // 128
_E2PAD = 1605632
_E2R = _E2PAD // 128
_F = 50000
_F2 = 53248
_FR = _F2 // 128
_H = 16

_NC, _NS = 2, 16
_MESH = plsc.VectorSubcoreMesh(
    core_axis_name="c", subcore_axis_name="s", num_cores=_NC, num_subcores=_NS)
_SC_PARAMS = pltpu.CompilerParams(use_tc_tiling_on_sc=False)


def _rs(d):
    return lax.rsqrt(jnp.maximum(d, 1e-12))


def _zero_fill(zero_v, n):
    def _z(i, c):
        zero_v[pl.ds(i * 16, 16)] = jnp.zeros((16,), F32)
        return c
    lax.fori_loop(0, n // 16, _z, 0)


def _ones_fill(ones_v):
    for i in range(8):
        ones_v[pl.ds(i * 16, 16)] = jnp.ones((16,), F32)


def _count_stream(idx_hbm, acc_sh, ones_v, idx_v, sem, base, ng):
    def _fire(parity):
        for j in range(8):
            pltpu.async_copy(ones_v, acc_sh.at[idx_v.at[parity].at[j]], sem,
                             add=True)

    def _drain(parity):
        for j in range(8):
            pltpu.make_async_copy(ones_v, acc_sh.at[idx_v.at[parity].at[j]],
                                  sem).wait()

    pltpu.sync_copy(idx_hbm.at[pl.ds(base, 8)], idx_v.at[0])
    _fire(0)

    def body(g, c):
        p = lax.rem(g, 2)
        q = 1 - p
        pltpu.sync_copy(idx_hbm.at[pl.ds(base + (g + 1) * 8, 8)], idx_v.at[q])
        _drain(p)
        _fire(q)
        return c
    lax.fori_loop(0, ng - 1, body, 0)
    _drain(lax.rem(ng - 1, 2))


def _s1a_body(d1r, xr, emb, deg1p, g0,
              deg1_sh, ones_v, zero_v, idx_v, gidx_v, grow_v, sem, semx):
    cid = lax.axis_index("c")
    sid = lax.axis_index("s")
    _ones_fill(ones_v)
    _zero_fill(zero_v, 640)
    pltpu.sync_copy(zero_v, deg1_sh.at[pl.ds(sid * 640, 640)])
    plsc.subcore_barrier()
    _count_stream(d1r, deg1_sh, ones_v, idx_v, sem,
                  cid * (_NS * 160) + sid * 160, 20)

    wid = sid * _NC + cid

    @pl.when(wid < 20)
    def _():
        base = wid * 4
        pltpu.sync_copy(xr.at[pl.ds(base, 4)], gidx_v)
        descs = [pltpu.async_copy(emb.at[gidx_v.at[j]], grow_v.at[j], semx)
                 for j in range(4)]
        for d in descs:
            d.wait()
        descs = [pltpu.async_copy(grow_v.at[j], g0.at[pl.ds((base + j) * 128, 128)],
                                  semx) for j in range(4)]
        for d in descs:
            d.wait()

    plsc.subcore_barrier()
    pltpu.sync_copy(deg1_sh.at[pl.ds(sid * 640, 640)],
                    deg1p.at[cid].at[pl.ds(sid * 640, 640)])


def _s1b_body(s2r, d2r, degap, degbp,
              dega_sh, degb_sh, ones_v, zero_v, idx_v, sem):
    cid = lax.axis_index("c")
    sid = lax.axis_index("s")
    _ones_fill(ones_v)
    _zero_fill(zero_v, 6400)
    pltpu.sync_copy(zero_v, dega_sh.at[pl.ds(sid * 6400, 6400)])
    pltpu.sync_copy(zero_v, degb_sh.at[pl.ds(sid * 6400, 6400)])
    plsc.subcore_barrier()
    base = cid * (_NS * 392) + sid * 392
    _count_stream(d2r, dega_sh, ones_v, idx_v, sem, base, 49)
    _count_stream(s2r, degb_sh, ones_v, idx_v, sem, base, 49)
    plsc.subcore_barrier()
    pltpu.sync_copy(dega_sh.at[pl.ds(sid * 6400, 6400)],
                    degap.at[cid].at[pl.ds(sid * 6400, 6400)])
    pltpu.sync_copy(degb_sh.at[pl.ds(sid * 6400, 6400)],
                    degbp.at[cid].at[pl.ds(sid * 6400, 6400)])


def _edge_pipeline(gir, sir, tbl, acc_sh, gi_v, si_v, rows_v, semg, sems,
                   semi, base, ng, k):
    def _fire_g(rp, ip):
        for j in range(k):
            pltpu.async_copy(tbl.at[gi_v.at[ip].at[j]], rows_v.at[rp].at[j], semg)

    def _wait_g(rp, ip):
        for j in range(k):
            pltpu.make_async_copy(tbl.at[gi_v.at[ip].at[j]], rows_v.at[rp].at[j],
                                  semg).wait()

    def _fire_s(rp, ip):
        for j in range(k):
            pltpu.async_copy(rows_v.at[rp].at[j], acc_sh.at[si_v.at[ip].at[j]],
                             sems, add=True)

    def _wait_s(rp, ip):
        for j in range(k):
            pltpu.make_async_copy(rows_v.at[rp].at[j], acc_sh.at[si_v.at[ip].at[j]],
                                  sems).wait()

    def _fire_i(g, slot):
        pltpu.async_copy(gir.at[pl.ds(base + g * k, k)], gi_v.at[slot], semi)
        pltpu.async_copy(sir.at[pl.ds(base + g * k, k)], si_v.at[slot], semi)

    def _wait_i(g, slot):
        pltpu.make_async_copy(gir.at[pl.ds(base + g * k, k)], gi_v.at[slot],
                              semi).wait()
        pltpu.make_async_copy(sir.at[pl.ds(base + g * k, k)], si_v.at[slot],
                              semi).wait()

    pltpu.sync_copy(gir.at[pl.ds(base, k)], gi_v.at[0])
    pltpu.sync_copy(sir.at[pl.ds(base, k)], si_v.at[0])
    _fire_g(0, 0)
    _fire_i(1, 1)

    def body(g, c):
        rp = lax.rem(g, 2)
        rq = 1 - rp
        ip = lax.rem(g, 3)
        iq = lax.rem(g + 1, 3)
        ir = lax.rem(g + 2, 3)
        _wait_i(g + 1, iq)
        _wait_g(rp, ip)
        _fire_g(rq, iq)

        @pl.when(g + 2 <= ng - 1)
        def _():
            _fire_i(g + 2, ir)
        _fire_s(rp, ip)
        _wait_s(rp, ip)
        return c
    lax.fori_loop(0, ng - 1, body, 0)
    last = ng - 1
    _wait_g(lax.rem(last, 2), lax.rem(last, 3))
    _fire_s(lax.rem(last, 2), lax.rem(last, 3))
    _wait_s(lax.rem(last, 2), lax.rem(last, 3))


def _conv1_body(s1r, d1r, hw1p, accp,
                acc_sh, si_v, di_v, rows_v, semg, sems, semi):
    cid = lax.axis_index("c")
    sid = lax.axis_index("s")
    pltpu.sync_copy(hw1p.at[pl.ds(sid * 640, 640)], acc_sh.at[pl.ds(sid * 640, 640)])
    plsc.subcore_barrier()
    base = cid * (_NS * 160) + sid * 160
    _edge_pipeline(s1r, d1r, hw1p, acc_sh, si_v, di_v, rows_v, semg, sems,
                   semi, base, 20, 8)
    plsc.subcore_barrier()
    pltpu.sync_copy(acc_sh.at[pl.ds(sid * 640, 640)],
                    accp.at[cid].at[pl.ds(sid * 640, 640)])


def _conv2_body(s2r, d2r, tbla, tblb, acca, accb,
                acc_sh, gi_v, si_v, rows_v, semg, sems, semi):
    cid = lax.axis_index("c")
    sid = lax.axis_index("s")
    base = sid * 784
    for gir, sir, tbl, out in ((s2r, d2r, tbla, acca), (d2r, s2r, tblb, accb)):
        pltpu.sync_copy(tbl.at[cid].at[pl.ds(sid * 6400, 6400)],
                        acc_sh.at[pl.ds(sid * 6400, 6400)])
        plsc.subcore_barrier()
        _edge_pipeline(gir, sir, tbl.at[cid], acc_sh, gi_v, si_v, rows_v,
                       semg, sems, semi, base, 196, 4)
        plsc.subcore_barrier()
        pltpu.sync_copy(acc_sh.at[pl.ds(sid * 6400, 6400)],
                        out.at[cid].at[pl.ds(sid * 6400, 6400)])
        plsc.subcore_barrier()


def _gather_pipeline(idxr, tbl, outr, idx_v, rows_v, semg, semw, base, ng, k):
    def _fire_g(p, off):
        for j in range(k):
            pltpu.async_copy(tbl.at[idx_v.at[p].at[j]], rows_v.at[p].at[j], semg)

    def _wait_g(p):
        for j in range(k):
            pltpu.make_async_copy(tbl.at[idx_v.at[p].at[j]], rows_v.at[p].at[j],
                                  semg).wait()

    def _fire_w(p, grp):
        for j in range(k):
            pltpu.async_copy(rows_v.at[p].at[j],
                             outr.at[pl.ds((base + grp * k + j) * 128, 128)],
                             semw)

    def _wait_w(p, grp):
        for j in range(k):
            pltpu.make_async_copy(rows_v.at[p].at[j],
                                  outr.at[pl.ds((base + grp * k + j) * 128, 128)],
                                  semw).wait()

    pltpu.sync_copy(idxr.at[pl.ds(base, k)], idx_v.at[0])
    _fire_g(0, 0)

    def body(g, c):
        p = lax.rem(g, 2)
        q = 1 - p
        pltpu.sync_copy(idxr.at[pl.ds(base + (g + 1) * k, k)], idx_v.at[q])
        _wait_g(p)
        _fire_g(q, g + 1)
        _fire_w(p, g)
        _wait_w(p, g)
        return c
    lax.fori_loop(0, ng - 1, body, 0)
    last = lax.rem(ng - 1, 2)
    _wait_g(last)
    _fire_w(last, ng - 1)
    _wait_w(last, ng - 1)


def _pair_body(pos0r, pos1r, h1, ga, gb, idx_v, rows_v, semg, semw):
    wid = lax.axis_index("s") * _NC + lax.axis_index("c")
    base = wid * 25
    _gather_pipeline(pos0r, h1, ga, idx_v, rows_v, semg, semw, base, 5, 5)
    _gather_pipeline(pos1r, h1, gb, idx_v, rows_v, semg, semw, base, 5, 5)


def _final_body(ier, ior, h2l, h2h, hel, heh, hol, hoh, idx_v, rows_v,
                semg, semw):
    wid = lax.axis_index("s") * _NC + lax.axis_index("c")
    base = wid * 13
    for idxr, tbl, outr in ((ier, h2l, hel), (ier, h2h, heh),
                            (ior, h2l, hol), (ior, h2h, hoh)):
        _gather_pipeline(idxr, tbl, outr, idx_v, rows_v, semg, semw, base, 2, 6)
        pltpu.sync_copy(idxr.at[pl.ds(base + 12, 1)], idx_v.at[0].at[pl.ds(0, 1)])
        pltpu.async_copy(tbl.at[idx_v.at[0].at[0]], rows_v.at[0].at[0], semg).wait()
        pltpu.async_copy(rows_v.at[0].at[0],
                         outr.at[pl.ds((base + 12) * 128, 128)], semw).wait()


def _t1_body(g0_ref, w_ref, deg_ref, o_ref):
    o_ref[...] = jnp.dot(g0_ref[...], w_ref[...], precision=HI) * _rs(deg_ref[...])


def _t4_body(accp_ref, hw_ref, deg_ref, b_ref, o_ref):
    acc = accp_ref[0] + accp_ref[1] - hw_ref[...]
    o_ref[...] = jax.nn.relu(_rs(deg_ref[...]) * acc + b_ref[...])


def _t5_body(ga_ref, gb_ref, wa_ref, wb_ref, dega_ref, degb_ref, oa_ref, ob_ref):
    hp = ga_ref[...] * gb_ref[...]
    ha = jnp.dot(hp * _rs(dega_ref[...]), wa_ref[...], precision=HI)
    hb = jnp.dot(hp * _rs(degb_ref[...]), wb_ref[...], precision=HI)
    oa_ref[0] = ha[:, :64]
    oa_ref[1] = ha[:, 64:]
    ob_ref[0] = hb[:, :64]
    ob_ref[1] = hb[:, 64:]


def _t6_body(acca_ref, accb_ref, dega_ref, degb_ref, bal_ref, bah_ref,
             bbl_ref, bbh_ref, ol_ref, oh_ref):
    rsa = _rs(dega_ref[...])
    rsb = _rs(degb_ref[...])
    ol_ref[...] = (jax.nn.relu(rsa * acca_ref[0] + bal_ref[...])
                   + jax.nn.relu(rsb * accb_ref[0] + bbl_ref[...]))
    oh_ref[...] = (jax.nn.relu(rsa * acca_ref[1] + bah_ref[...])
                   + jax.nn.relu(rsb * accb_ref[1] + bbh_ref[...]))


def _t7_body(hel_ref, hol_ref, heh_ref, hoh_ref, wl_ref, wh_ref, s_ref, b_ref,
             o_ref):
    prod = jnp.concatenate([hel_ref[...] * hol_ref[...] * wl_ref[...],
                            heh_ref[...] * hoh_ref[...] * wh_ref[...]], axis=1)
    o_ref[...] = jnp.dot(prod, s_ref[...], precision=HI) + b_ref[...]


def _sds(shape, dtype=F32):
    return jax.ShapeDtypeStruct(shape, dtype)


_s1a_call = pl.kernel(
    _s1a_body,
    out_type=(_sds((2, _N2)), _sds((_N2, 128))),
    mesh=_MESH,
    compiler_params=_SC_PARAMS,
    scratch_types=[
        pltpu.VMEM_SHARED((_N2,), F32),
        pltpu.VMEM((128,), F32),
        pltpu.VMEM((640,), F32),
        pltpu.VMEM((2, 8, 128), I32),
        pltpu.VMEM((4, 128), I32),
        pltpu.VMEM((4, 128, 128), F32),
        pltpu.SemaphoreType.DMA,
        pltpu.SemaphoreType.DMA,
    ])

_s1b_call = pl.kernel(
    _s1b_body,
    out_type=(_sds((2, _P2)), _sds((2, _P2))),
    mesh=_MESH,
    compiler_params=_SC_PARAMS,
    scratch_types=[
        pltpu.VMEM_SHARED((_P2,), F32),
        pltpu.VMEM_SHARED((_P2,), F32),
        pltpu.VMEM((128,), F32),
        pltpu.VMEM((6400,), F32),
        pltpu.VMEM((2, 8, 128), I32),
        pltpu.SemaphoreType.DMA,
    ])

_conv1_call = pl.kernel(
    _conv1_body,
    out_type=_sds((2, _N2, 32)),
    mesh=_MESH,
    compiler_params=_SC_PARAMS,
    scratch_types=[
        pltpu.VMEM_SHARED((_N2, 32), F32),
        pltpu.VMEM((3, 8, 128), I32),
        pltpu.VMEM((3, 8, 128), I32),
        pltpu.VMEM((2, 8, 128, 32), F32),
        pltpu.SemaphoreType.DMA,
        pltpu.SemaphoreType.DMA,
        pltpu.SemaphoreType.DMA,
    ])

_conv2_call = pl.kernel(
    _conv2_body,
    out_type=(_sds((2, _P2, _H)), _sds((2, _P2, _H))),
    mesh=_MESH,
    compiler_params=_SC_PARAMS,
    scratch_types=[
        pltpu.VMEM_SHARED((_P2, _H), F32),
        pltpu.VMEM((3, 4, 128), I32),
        pltpu.VMEM((3, 4, 128), I32),
        pltpu.VMEM((2, 4, 128, _H), F32),
        pltpu.SemaphoreType.DMA,
        pltpu.SemaphoreType.DMA,
        pltpu.SemaphoreType.DMA,
    ])

_pair_call = pl.kernel(
    _pair_body,
    out_type=(_sds((_P2, 32)), _sds((_P2, 32))),
    mesh=_MESH,
    compiler_params=_SC_PARAMS,
    scratch_types=[
        pltpu.VMEM((2, 5, 128), I32),
        pltpu.VMEM((2, 5, 128, 32), F32),
        pltpu.SemaphoreType.DMA,
        pltpu.SemaphoreType.DMA,
    ])

_final_call = pl.kernel(
    _final_body,
    out_type=(_sds((_F2, _H)), _sds((_F2, _H)), _sds((_F2, _H)),
              _sds((_F2, _H))),
    mesh=_MESH,
    compiler_params=_SC_PARAMS,
    scratch_types=[
        pltpu.VMEM((2, 6, 128), I32),
        pltpu.VMEM((2, 6, 128, _H), F32),
        pltpu.SemaphoreType.DMA,
        pltpu.SemaphoreType.DMA,
    ])

_t1_call = pl.pallas_call(
    _t1_body, grid=(10,),
    in_specs=[pl.BlockSpec((1024, 128), lambda i: (i, 0)),
              pl.BlockSpec((128, 32), lambda i: (0, 0)),
              pl.BlockSpec((1024, 1), lambda i: (i, 0))],
    out_specs=pl.BlockSpec((1024, 32), lambda i: (i, 0)),
    out_shape=_sds((_N2, 32)))

_t4_call = pl.pallas_call(
    _t4_body, grid=(10,),
    in_specs=[pl.BlockSpec((2, 1024, 32), lambda i: (0, i, 0)),
              pl.BlockSpec((1024, 32), lambda i: (i, 0)),
              pl.BlockSpec((1024, 1), lambda i: (i, 0)),
              pl.BlockSpec((1, 32), lambda i: (0, 0))],
    out_specs=pl.BlockSpec((1024, 32), lambda i: (i, 0)),
    out_shape=_sds((_N2, 32)))

_t5_call = pl.pallas_call(
    _t5_body, grid=(25,),
    in_specs=[pl.BlockSpec((1024, 128), lambda i: (i, 0)),
              pl.BlockSpec((1024, 128), lambda i: (i, 0)),
              pl.BlockSpec((128, 128), lambda i: (0, 0)),
              pl.BlockSpec((128, 128), lambda i: (0, 0)),
              pl.BlockSpec((1024, 128), lambda i: (i, 0)),
              pl.BlockSpec((1024, 128), lambda i: (i, 0))],
    out_specs=(pl.BlockSpec((2, 1024, 64), lambda i: (0, i, 0)),
               pl.BlockSpec((2, 1024, 64), lambda i: (0, i, 0))),
    out_shape=(_sds((2, _P2 // 4, 64)), _sds((2, _P2 // 4, 64))))

_t6_call = pl.pallas_call(
    _t6_body, grid=(25,),
    in_specs=[pl.BlockSpec((2, 1024, 64), lambda i: (0, i, 0)),
              pl.BlockSpec((2, 1024, 64), lambda i: (0, i, 0)),
              pl.BlockSpec((1024, 64), lambda i: (i, 0)),
              pl.BlockSpec((1024, 64), lambda i: (i, 0)),
              pl.BlockSpec((1, 64), lambda i: (0, 0)),
              pl.BlockSpec((1, 64), lambda i: (0, 0)),
              pl.BlockSpec((1, 64), lambda i: (0, 0)),
              pl.BlockSpec((1, 64), lambda i: (0, 0))],
    out_specs=(pl.BlockSpec((1024, 64), lambda i: (i, 0)),
               pl.BlockSpec((1024, 64), lambda i: (i, 0))),
    out_shape=(_sds((_P2 // 4, 64)), _sds((_P2 // 4, 64))))

_t7_call = pl.pallas_call(
    _t7_body, grid=(13,),
    in_specs=[pl.BlockSpec((1024, 64), lambda i: (i, 0)),
              pl.BlockSpec((1024, 64), lambda i: (i, 0)),
              pl.BlockSpec((1024, 64), lambda i: (i, 0)),
              pl.BlockSpec((1024, 64), lambda i: (i, 0)),
              pl.BlockSpec((1, 64), lambda i: (0, 0)),
              pl.BlockSpec((1, 64), lambda i: (0, 0)),
              pl.BlockSpec((128, 4), lambda i: (0, 0)),
              pl.BlockSpec((1, 1), lambda i: (0, 0))],
    out_specs=pl.BlockSpec((1024, 4), lambda i: (i, 0)),
    out_shape=_sds((_F2 // 4, 4)))


def _pad_spread(a, n, base, mod):
    fill = base + (jnp.arange(n - a.shape[0], dtype=I32) % mod)
    return jnp.concatenate([a.astype(I32), fill])


def kernel(x, edge1, pos, idx, ei2, emb, W1, b1, W2a, b2a, W2b, b2b, Wp, bp):
    s1r = _pad_spread(edge1[0], _E1PAD, _N, _N2 - _N).reshape(_E1R, 128)
    d1r = _pad_spread(edge1[1], _E1PAD, _N, _N2 - _N).reshape(_E1R, 128)
    s2r = _pad_spread(ei2[0], _E2PAD, _P, _P2 - _P).reshape(_E2R, 128)
    d2r = _pad_spread(ei2[1], _E2PAD, _P, _P2 - _P).reshape(_E2R, 128)
    xr = _pad_spread(x, _N2, 0, _N).reshape(_NR, 128)
    pos0r = _pad_spread(pos[:, 0], _P2, 0, _N).reshape(_PR, 128)
    pos1r = _pad_spread(pos[:, 1], _P2, 0, _N).reshape(_PR, 128)
    idx2 = idx.astype(I32).reshape(_F, 2)
    ier = _pad_spread(idx2[:, 0], _F2, 0, _P).reshape(_FR, 128)
    ior = _pad_spread(idx2[:, 1], _F2, 0, _P).reshape(_FR, 128)

    deg1p, g0 = _s1a_call(d1r, xr, emb)
    deg1c = (deg1p[0] + deg1p[1] + 1.0).reshape(_N2, 1)

    hw1p = _t1_call(g0, W1, deg1c)
    accp = _conv1_call(s1r, d1r, hw1p)
    h1 = _t4_call(accp, hw1p, deg1c, b1.reshape(1, 32))

    degap, degbp = _s1b_call(s2r, d2r)
    degac = (degap[0] + degap[1] + 1.0).reshape(_P2, 1)
    degbc = (degbp[0] + degbp[1] + 1.0).reshape(_P2, 1)
    dega4 = jnp.broadcast_to(degac, (_P2, 32)).reshape(_P2 // 4, 128)
    degb4 = jnp.broadcast_to(degbc, (_P2, 32)).reshape(_P2 // 4, 128)
    dega2 = jnp.broadcast_to(degac, (_P2, _H)).reshape(_P2 // 4, 64)
    degb2 = jnp.broadcast_to(degbc, (_P2, _H)).reshape(_P2 // 4, 64)

    ga, gb = _pair_call(pos0r, pos1r, h1)
    eye4 = jnp.eye(4, dtype=F32)
    wacat = jnp.concatenate([jnp.kron(eye4, W2a[:, :_H]),
                             jnp.kron(eye4, W2a[:, _H:])], axis=1)
    wbcat = jnp.concatenate([jnp.kron(eye4, W2b[:, :_H]),
                             jnp.kron(eye4, W2b[:, _H:])], axis=1)

    hap2, hbp2 = _t5_call(ga.reshape(_P2 // 4, 128), gb.reshape(_P2 // 4, 128),
                          wacat, wbcat, dega4, degb4)
    acca2, accb2 = _conv2_call(s2r, d2r, hap2.reshape(2, _P2, _H),
                               hbp2.reshape(2, _P2, _H))
    h2l, h2h = _t6_call(acca2.reshape(2, _P2 // 4, 64),
                        accb2.reshape(2, _P2 // 4, 64), dega2, degb2,
                        jnp.tile(b2a[:_H], 4).reshape(1, 64),
                        jnp.tile(b2a[_H:], 4).reshape(1, 64),
                        jnp.tile(b2b[:_H], 4).reshape(1, 64),
                        jnp.tile(b2b[_H:], 4).reshape(1, 64))

    hel, heh, hol, hoh = _final_call(ier, ior, h2l.reshape(_P2, _H),
                                     h2h.reshape(_P2, _H))
    sel = jax.nn.one_hot((jnp.arange(128) % 64) // _H, 4, dtype=F32)
    out4 = _t7_call(hel.reshape(_F2 // 4, 64), hol.reshape(_F2 // 4, 64),
                    heh.reshape(_F2 // 4, 64), hoh.reshape(_F2 // 4, 64),
                    jnp.tile(Wp[:_H, 0], 4).reshape(1, 64),
                    jnp.tile(Wp[_H:, 0], 4).reshape(1, 64),
                    sel, bp.reshape(1, 1))
    return out4.reshape(_F2, 1)[:_F]


# speedup vs baseline: 71.0459x; 1.0001x over previous
import functools

import jax
import jax.numpy as jnp
from jax import lax
from jax.experimental import pallas as pl
from jax.experimental.pallas import tpu as pltpu
from jax.experimental.pallas import tpu_sc as plsc

F32 = jnp.float32
I32 = jnp.int32
HI = lax.Precision.HIGHEST

_N = 10000
_N2 = 10240
_NR = _N2 // 128
_E1PAD = 655360
_E1R = _E1PAD // 128
_P = 100000
_P2 = 102400
_PR = _P2 // 128
_E2PAD = 1605632
_E2R = _E2PAD // 128
_F = 50000
_F2 = 53248
_FR = _F2 // 128
_H = 16

_NC, _NS = 2, 16
_SC_PARAMS = pltpu.CompilerParams(use_tc_tiling_on_sc=False)


def _rs(d):
    return lax.rsqrt(jnp.maximum(d, 1e-12))


def _zero_fill(zero_v, n):
    def _z(i, c):
        zero_v[pl.ds(i * 16, 16)] = jnp.zeros((16,), F32)
        return c
    lax.fori_loop(0, n // 16, _z, 0)


def _ones_fill(ones_v):
    for i in range(8):
        ones_v[pl.ds(i * 16, 16)] = jnp.ones((16,), F32)


def _count_stream(idx_hbm, acc_sh, ones_v, idx_v, sem, base, ng):
    def _fire(parity):
        for j in range(8):
            pltpu.async_copy(ones_v, acc_sh.at[idx_v.at[parity].at[j]], sem,
                             add=True)

    def _drain(parity):
        for j in range(8):
            pltpu.make_async_copy(ones_v, acc_sh.at[idx_v.at[parity].at[j]],
                                  sem).wait()

    pltpu.sync_copy(idx_hbm.at[pl.ds(base, 8)], idx_v.at[0])
    _fire(0)

    def body(g, c):
        p = lax.rem(g, 2)
        q = 1 - p
        pltpu.sync_copy(idx_hbm.at[pl.ds(base + (g + 1) * 8, 8)], idx_v.at[q])
        _drain(p)
        _fire(q)
        return c
    lax.fori_loop(0, ng - 1, body, 0)
    _drain(lax.rem(ng - 1, 2))


def _s1a_body(d1r, xr, emb, deg1p, g0,
              deg1_sh, ones_v, zero_v, idx_v, gidx_v, grow_v, sem, semx):
    cid = lax.axis_index("c")
    sid = lax.axis_index("s")
    _ones_fill(ones_v)
    _zero_fill(zero_v, 640)
    pltpu.sync_copy(zero_v, deg1_sh.at[pl.ds(sid * 640, 640)])
    plsc.subcore_barrier()
    _count_stream(d1r, deg1_sh, ones_v, idx_v, sem,
                  cid * (_NS * 160) + sid * 160, 20)

    wid = sid * _NC + cid

    @pl.when(wid < 20)
    def _():
        base = wid * 4
        pltpu.sync_copy(xr.at[pl.ds(base, 4)], gidx_v)
        descs = [pltpu.async_copy(emb.at[gidx_v.at[j]], grow_v.at[j], semx)
                 for j in range(4)]
        for d in descs:
            d.wait()
        descs = [pltpu.async_copy(grow_v.at[j], g0.at[pl.ds((base + j) * 128, 128)],
                                  semx) for j in range(4)]
        for d in descs:
            d.wait()

    plsc.subcore_barrier()
    pltpu.sync_copy(deg1_sh.at[pl.ds(sid * 640, 640)],
                    deg1p.at[cid].at[pl.ds(sid * 640, 640)])


def _s1b_body(s2r, d2r, degap, degbp,
              dega_sh, degb_sh, ones_v, zero_v, idx_v, sem):
    cid = lax.axis_index("c")
    sid = lax.axis_index("s")
    _ones_fill(ones_v)
    _zero_fill(zero_v, 6400)
    pltpu.sync_copy(zero_v, dega_sh.at[pl.ds(sid * 6400, 6400)])
    pltpu.sync_copy(zero_v, degb_sh.at[pl.ds(sid * 6400, 6400)])
    plsc.subcore_barrier()
    base = cid * (_NS * 392) + sid * 392
    _count_stream(d2r, dega_sh, ones_v, idx_v, sem, base, 49)
    _count_stream(s2r, degb_sh, ones_v, idx_v, sem, base, 49)
    plsc.subcore_barrier()
    pltpu.sync_copy(dega_sh.at[pl.ds(sid * 6400, 6400)],
                    degap.at[cid].at[pl.ds(sid * 6400, 6400)])
    pltpu.sync_copy(degb_sh.at[pl.ds(sid * 6400, 6400)],
                    degbp.at[cid].at[pl.ds(sid * 6400, 6400)])


def _edge_pipeline(gir, sir, tbl, acc_sh, gi_v, si_v, rows_v, semg, sems,
                   semi, base, ng, k):
    def _fire_g(rp, ip):
        for j in range(k):
            pltpu.async_copy(tbl.at[gi_v.at[ip].at[j]], rows_v.at[rp].at[j], semg)

    def _wait_g(rp, ip):
        for j in range(k):
            pltpu.make_async_copy(tbl.at[gi_v.at[ip].at[j]], rows_v.at[rp].at[j],
                                  semg).wait()

    def _fire_s(rp, ip):
        for j in range(k):
            pltpu.async_copy(rows_v.at[rp].at[j], acc_sh.at[si_v.at[ip].at[j]],
                             sems, add=True)

    def _wait_s(rp, ip):
        for j in range(k):
            pltpu.make_async_copy(rows_v.at[rp].at[j], acc_sh.at[si_v.at[ip].at[j]],
                                  sems).wait()

    def _fire_i(g, slot):
        pltpu.async_copy(gir.at[pl.ds(base + g * k, k)], gi_v.at[slot], semi)
        pltpu.async_copy(sir.at[pl.ds(base + g * k, k)], si_v.at[slot], semi)

    def _wait_i(g, slot):
        pltpu.make_async_copy(gir.at[pl.ds(base + g * k, k)], gi_v.at[slot],
                              semi).wait()
        pltpu.make_async_copy(sir.at[pl.ds(base + g * k, k)], si_v.at[slot],
                              semi).wait()

    pltpu.sync_copy(gir.at[pl.ds(base, k)], gi_v.at[0])
    pltpu.sync_copy(sir.at[pl.ds(base, k)], si_v.at[0])
    _fire_g(0, 0)
    _fire_i(1, 1)

    def body(g, c):
        rp = lax.rem(g, 2)
        rq = 1 - rp
        ip = lax.rem(g, 3)
        iq = lax.rem(g + 1, 3)
        ir = lax.rem(g + 2, 3)
        _wait_i(g + 1, iq)
        _wait_g(rp, ip)
        _fire_g(rq, iq)

        @pl.when(g + 2 <= ng - 1)
        def _():
            _fire_i(g + 2, ir)
        _fire_s(rp, ip)
        _wait_s(rp, ip)
        return c
    lax.fori_loop(0, ng - 1, body, 0)
    last = ng - 1
    _wait_g(lax.rem(last, 2), lax.rem(last, 3))
    _fire_s(lax.rem(last, 2), lax.rem(last, 3))
    _wait_s(lax.rem(last, 2), lax.rem(last, 3))


def _conv1_body(s1r, d1r, hw1p, accp,
                acc_sh, si_v, di_v, rows_v, semg, sems, semi):
    cid = lax.axis_index("c")
    sid = lax.axis_index("s")
    pltpu.sync_copy(hw1p.at[pl.ds(sid * 640, 640)], acc_sh.at[pl.ds(sid * 640, 640)])
    plsc.subcore_barrier()
    base = cid * (_NS * 160) + sid * 160
    _edge_pipeline(s1r, d1r, hw1p, acc_sh, si_v, di_v, rows_v, semg, sems,
                   semi, base, 20, 8)
    plsc.subcore_barrier()
    pltpu.sync_copy(acc_sh.at[pl.ds(sid * 640, 640)],
                    accp.at[cid].at[pl.ds(sid * 640, 640)])


def _conv2_body(s2r, d2r, tbla, tblb, acca, accb,
                acc_sh, gi_v, si_v, rows_v, semg, sems, semi):
    cid = lax.axis_index("c")
    sid = lax.axis_index("s")
    base = sid * 784
    for gir, sir, tbl, out in ((s2r, d2r, tbla, acca), (d2r, s2r, tblb, accb)):
        pltpu.sync_copy(tbl.at[cid].at[pl.ds(sid * 6400, 6400)],
                        acc_sh.at[pl.ds(sid * 6400, 6400)])
        plsc.subcore_barrier()
        _edge_pipeline(gir, sir, tbl.at[cid], acc_sh, gi_v, si_v, rows_v,
                       semg, sems, semi, base, 196, 4)
        plsc.subcore_barrier()
        pltpu.sync_copy(acc_sh.at[pl.ds(sid * 6400, 6400)],
                        out.at[cid].at[pl.ds(sid * 6400, 6400)])
        plsc.subcore_barrier()


def _gather_pipeline(idxr, tbl, outr, idx_v, rows_v, semg, semw, base, ng, k):
    def _fire_g(p, off):
        for j in range(k):
            pltpu.async_copy(tbl.at[idx_v.at[p].at[j]], rows_v.at[p].at[j], semg)

    def _wait_g(p):
        for j in range(k):
            pltpu.make_async_copy(tbl.at[idx_v.at[p].at[j]], rows_v.at[p].at[j],
                                  semg).wait()

    def _fire_w(p, grp):
        for j in range(k):
            pltpu.async_copy(rows_v.at[p].at[j],
                             outr.at[pl.ds((base + grp * k + j) * 128, 128)],
                             semw)

    def _wait_w(p, grp):
        for j in range(k):
            pltpu.make_async_copy(rows_v.at[p].at[j],
                                  outr.at[pl.ds((base + grp * k + j) * 128, 128)],
                                  semw).wait()

    pltpu.sync_copy(idxr.at[pl.ds(base, k)], idx_v.at[0])
    _fire_g(0, 0)

    def body(g, c):
        p = lax.rem(g, 2)
        q = 1 - p
        pltpu.sync_copy(idxr.at[pl.ds(base + (g + 1) * k, k)], idx_v.at[q])
        _wait_g(p)
        _fire_g(q, g + 1)
        _fire_w(p, g)
        _wait_w(p, g)
        return c
    lax.fori_loop(0, ng - 1, body, 0)
    last = lax.rem(ng - 1, 2)
    _wait_g(last)
    _fire_w(last, ng - 1)
    _wait_w(last, ng - 1)


def _pair_body(pos0r, pos1r, h1, ga, gb, idx_v, rows_v, semg, semw):
    wid = lax.axis_index("s") * _NC + lax.axis_index("c")
    base = wid * 25
    _gather_pipeline(pos0r, h1, ga, idx_v, rows_v, semg, semw, base, 5, 5)
    _gather_pipeline(pos1r, h1, gb, idx_v, rows_v, semg, semw, base, 5, 5)


def _final_body(ier, ior, h2l, h2h, hel, heh, hol, hoh, idx_v, rows_v,
                semg, semw):
    wid = lax.axis_index("s") * _NC + lax.axis_index("c")
    base = wid * 13
    for idxr, tbl, outr in ((ier, h2l, hel), (ier, h2h, heh),
                            (ior, h2l, hol), (ior, h2h, hoh)):
        _gather_pipeline(idxr, tbl, outr, idx_v, rows_v, semg, semw, base, 2, 6)
        pltpu.sync_copy(idxr.at[pl.ds(base + 12, 1)], idx_v.at[0].at[pl.ds(0, 1)])
        pltpu.async_copy(tbl.at[idx_v.at[0].at[0]], rows_v.at[0].at[0], semg).wait()
        pltpu.async_copy(rows_v.at[0].at[0],
                         outr.at[pl.ds((base + 12) * 128, 128)], semw).wait()


def _t1_body(g0_ref, w_ref, deg_ref, o_ref):
    o_ref[...] = jnp.dot(g0_ref[...], w_ref[...], precision=HI) * _rs(deg_ref[...])


def _t4_body(accp_ref, hw_ref, deg_ref, b_ref, o_ref):
    acc = accp_ref[0] + accp_ref[1] - hw_ref[...]
    o_ref[...] = jax.nn.relu(_rs(deg_ref[...]) * acc + b_ref[...])


def _t5_body(ga_ref, gb_ref, wa_ref, wb_ref, dega_ref, degb_ref, oa_ref, ob_ref):
    hp = ga_ref[...] * gb_ref[...]
    ha = jnp.dot(hp * _rs(dega_ref[...]), wa_ref[...], precision=HI)
    hb = jnp.dot(hp * _rs(degb_ref[...]), wb_ref[...], precision=HI)
    oa_ref[0] = ha[:, :64]
    oa_ref[1] = ha[:, 64:]
    ob_ref[0] = hb[:, :64]
    ob_ref[1] = hb[:, 64:]


def _t6_body(acca_ref, accb_ref, dega_ref, degb_ref, bal_ref, bah_ref,
             bbl_ref, bbh_ref, ol_ref, oh_ref):
    rsa = _rs(dega_ref[...])
    rsb = _rs(degb_ref[...])
    ol_ref[...] = (jax.nn.relu(rsa * acca_ref[0] + bal_ref[...])
                   + jax.nn.relu(rsb * accb_ref[0] + bbl_ref[...]))
    oh_ref[...] = (jax.nn.relu(rsa * acca_ref[1] + bah_ref[...])
                   + jax.nn.relu(rsb * accb_ref[1] + bbh_ref[...]))


def _t7_body(hel_ref, hol_ref, heh_ref, hoh_ref, wl_ref, wh_ref, s_ref, b_ref,
             o_ref):
    prod = jnp.concatenate([hel_ref[...] * hol_ref[...] * wl_ref[...],
                            heh_ref[...] * hoh_ref[...] * wh_ref[...]], axis=1)
    o_ref[...] = jnp.dot(prod, s_ref[...], precision=HI) + b_ref[...]


def _sds(shape, dtype=F32):
    return jax.ShapeDtypeStruct(shape, dtype)


@functools.cache
def _sc_calls():
    mesh = plsc.VectorSubcoreMesh(
        core_axis_name="c", subcore_axis_name="s", num_cores=_NC,
        num_subcores=_NS)

    _s1a_call = pl.kernel(
        _s1a_body,
        out_type=(_sds((2, _N2)), _sds((_N2, 128))),
        mesh=mesh,
        compiler_params=_SC_PARAMS,
        scratch_types=[
            pltpu.VMEM_SHARED((_N2,), F32),
            pltpu.VMEM((128,), F32),
            pltpu.VMEM((640,), F32),
            pltpu.VMEM((2, 8, 128), I32),
            pltpu.VMEM((4, 128), I32),
            pltpu.VMEM((4, 128, 128), F32),
            pltpu.SemaphoreType.DMA,
            pltpu.SemaphoreType.DMA,
        ])

    _s1b_call = pl.kernel(
        _s1b_body,
        out_type=(_sds((2, _P2)), _sds((2, _P2))),
        mesh=mesh,
        compiler_params=_SC_PARAMS,
        scratch_types=[
            pltpu.VMEM_SHARED((_P2,), F32),
            pltpu.VMEM_SHARED((_P2,), F32),
            pltpu.VMEM((128,), F32),
            pltpu.VMEM((6400,), F32),
            pltpu.VMEM((2, 8, 128), I32),
            pltpu.SemaphoreType.DMA,
        ])

    _conv1_call = pl.kernel(
        _conv1_body,
        out_type=_sds((2, _N2, 32)),
        mesh=mesh,
        compiler_params=_SC_PARAMS,
        scratch_types=[
            pltpu.VMEM_SHARED((_N2, 32), F32),
            pltpu.VMEM((3, 8, 128), I32),
            pltpu.VMEM((3, 8, 128), I32),
            pltpu.VMEM((2, 8, 128, 32), F32),
            pltpu.SemaphoreType.DMA,
            pltpu.SemaphoreType.DMA,
            pltpu.SemaphoreType.DMA,
        ])

    _conv2_call = pl.kernel(
        _conv2_body,
        out_type=(_sds((2, _P2, _H)), _sds((2, _P2, _H))),
        mesh=mesh,
        compiler_params=_SC_PARAMS,
        scratch_types=[
            pltpu.VMEM_SHARED((_P2, _H), F32),
            pltpu.VMEM((3, 4, 128), I32),
            pltpu.VMEM((3, 4, 128), I32),
            pltpu.VMEM((2, 4, 128, _H), F32),
            pltpu.SemaphoreType.DMA,
            pltpu.SemaphoreType.DMA,
            pltpu.SemaphoreType.DMA,
        ])

    _pair_call = pl.kernel(
        _pair_body,
        out_type=(_sds((_P2, 32)), _sds((_P2, 32))),
        mesh=mesh,
        compiler_params=_SC_PARAMS,
        scratch_types=[
            pltpu.VMEM((2, 5, 128), I32),
            pltpu.VMEM((2, 5, 128, 32), F32),
            pltpu.SemaphoreType.DMA,
            pltpu.SemaphoreType.DMA,
        ])

    _final_call = pl.kernel(
        _final_body,
        out_type=(_sds((_F2, _H)), _sds((_F2, _H)), _sds((_F2, _H)),
                  _sds((_F2, _H))),
        mesh=mesh,
        compiler_params=_SC_PARAMS,
        scratch_types=[
            pltpu.VMEM((2, 6, 128), I32),
            pltpu.VMEM((2, 6, 128, _H), F32),
            pltpu.SemaphoreType.DMA,
            pltpu.SemaphoreType.DMA,
        ])
    return _s1a_call, _s1b_call, _conv1_call, _conv2_call, _pair_call, _final_call


_t1_call = pl.pallas_call(
    _t1_body, grid=(10,),
    in_specs=[pl.BlockSpec((1024, 128), lambda i: (i, 0)),
              pl.BlockSpec((128, 32), lambda i: (0, 0)),
              pl.BlockSpec((1024, 1), lambda i: (i, 0))],
    out_specs=pl.BlockSpec((1024, 32), lambda i: (i, 0)),
    out_shape=_sds((_N2, 32)))

_t4_call = pl.pallas_call(
    _t4_body, grid=(10,),
    in_specs=[pl.BlockSpec((2, 1024, 32), lambda i: (0, i, 0)),
              pl.BlockSpec((1024, 32), lambda i: (i, 0)),
              pl.BlockSpec((1024, 1), lambda i: (i, 0)),
              pl.BlockSpec((1, 32), lambda i: (0, 0))],
    out_specs=pl.BlockSpec((1024, 32), lambda i: (i, 0)),
    out_shape=_sds((_N2, 32)))

_t5_call = pl.pallas_call(
    _t5_body, grid=(25,),
    in_specs=[pl.BlockSpec((1024, 128), lambda i: (i, 0)),
              pl.BlockSpec((1024, 128), lambda i: (i, 0)),
              pl.BlockSpec((128, 128), lambda i: (0, 0)),
              pl.BlockSpec((128, 128), lambda i: (0, 0)),
              pl.BlockSpec((1024, 128), lambda i: (i, 0)),
              pl.BlockSpec((1024, 128), lambda i: (i, 0))],
    out_specs=(pl.BlockSpec((2, 1024, 64), lambda i: (0, i, 0)),
               pl.BlockSpec((2, 1024, 64), lambda i: (0, i, 0))),
    out_shape=(_sds((2, _P2 // 4, 64)), _sds((2, _P2 // 4, 64))))

_t6_call = pl.pallas_call(
    _t6_body, grid=(25,),
    in_specs=[pl.BlockSpec((2, 1024, 64), lambda i: (0, i, 0)),
              pl.BlockSpec((2, 1024, 64), lambda i: (0, i, 0)),
              pl.BlockSpec((1024, 64), lambda i: (i, 0)),
              pl.BlockSpec((1024, 64), lambda i: (i, 0)),
              pl.BlockSpec((1, 64), lambda i: (0, 0)),
              pl.BlockSpec((1, 64), lambda i: (0, 0)),
              pl.BlockSpec((1, 64), lambda i: (0, 0)),
              pl.BlockSpec((1, 64), lambda i: (0, 0))],
    out_specs=(pl.BlockSpec((1024, 64), lambda i: (i, 0)),
               pl.BlockSpec((1024, 64), lambda i: (i, 0))),
    out_shape=(_sds((_P2 // 4, 64)), _sds((_P2 // 4, 64))))

_t7_call = pl.pallas_call(
    _t7_body, grid=(13,),
    in_specs=[pl.BlockSpec((1024, 64), lambda i: (i, 0)),
              pl.BlockSpec((1024, 64), lambda i: (i, 0)),
              pl.BlockSpec((1024, 64), lambda i: (i, 0)),
              pl.BlockSpec((1024, 64), lambda i: (i, 0)),
              pl.BlockSpec((1, 64), lambda i: (0, 0)),
              pl.BlockSpec((1, 64), lambda i: (0, 0)),
              pl.BlockSpec((128, 4), lambda i: (0, 0)),
              pl.BlockSpec((1, 1), lambda i: (0, 0))],
    out_specs=pl.BlockSpec((1024, 4), lambda i: (i, 0)),
    out_shape=_sds((_F2 // 4, 4)))


def _pad_spread(a, n, base, mod):
    fill = base + (jnp.arange(n - a.shape[0], dtype=I32) % mod)
    return jnp.concatenate([a.astype(I32), fill])


def kernel(x, edge1, pos, idx, ei2, emb, W1, b1, W2a, b2a, W2b, b2b, Wp, bp):
    s1r = _pad_spread(edge1[0], _E1PAD, _N, _N2 - _N).reshape(_E1R, 128)
    d1r = _pad_spread(edge1[1], _E1PAD, _N, _N2 - _N).reshape(_E1R, 128)
    s2r = _pad_spread(ei2[0], _E2PAD, _P, _P2 - _P).reshape(_E2R, 128)
    d2r = _pad_spread(ei2[1], _E2PAD, _P, _P2 - _P).reshape(_E2R, 128)
    xr = _pad_spread(x, _N2, 0, _N).reshape(_NR, 128)
    pos0r = _pad_spread(pos[:, 0], _P2, 0, _N).reshape(_PR, 128)
    pos1r = _pad_spread(pos[:, 1], _P2, 0, _N).reshape(_PR, 128)
    idx2 = idx.astype(I32).reshape(_F, 2)
    ier = _pad_spread(idx2[:, 0], _F2, 0, _P).reshape(_FR, 128)
    ior = _pad_spread(idx2[:, 1], _F2, 0, _P).reshape(_FR, 128)

    (_s1a_call, _s1b_call, _conv1_call, _conv2_call, _pair_call,
     _final_call) = _sc_calls()

    deg1p, g0 = _s1a_call(d1r, xr, emb)
    deg1c = (deg1p[0] + deg1p[1] + 1.0).reshape(_N2, 1)

    hw1p = _t1_call(g0, W1, deg1c)
    accp = _conv1_call(s1r, d1r, hw1p)
    h1 = _t4_call(accp, hw1p, deg1c, b1.reshape(1, 32))

    degap, degbp = _s1b_call(s2r, d2r)
    degac = (degap[0] + degap[1] + 1.0).reshape(_P2, 1)
    degbc = (degbp[0] + degbp[1] + 1.0).reshape(_P2, 1)
    dega4 = jnp.broadcast_to(degac, (_P2, 32)).reshape(_P2 // 4, 128)
    degb4 = jnp.broadcast_to(degbc, (_P2, 32)).reshape(_P2 // 4, 128)
    dega2 = jnp.broadcast_to(degac, (_P2, _H)).reshape(_P2 // 4, 64)
    degb2 = jnp.broadcast_to(degbc, (_P2, _H)).reshape(_P2 // 4, 64)

    ga, gb = _pair_call(pos0r, pos1r, h1)
    eye4 = jnp.eye(4, dtype=F32)
    wacat = jnp.concatenate([jnp.kron(eye4, W2a[:, :_H]),
                             jnp.kron(eye4, W2a[:, _H:])], axis=1)
    wbcat = jnp.concatenate([jnp.kron(eye4, W2b[:, :_H]),
                             jnp.kron(eye4, W2b[:, _H:])], axis=1)

    hap2, hbp2 = _t5_call(ga.reshape(_P2 // 4, 128), gb.reshape(_P2 // 4, 128),
                          wacat, wbcat, dega4, degb4)
    acca2, accb2 = _conv2_call(s2r, d2r, hap2.reshape(2, _P2, _H),
                               hbp2.reshape(2, _P2, _H))
    h2l, h2h = _t6_call(acca2.reshape(2, _P2 // 4, 64),
                        accb2.reshape(2, _P2 // 4, 64), dega2, degb2,
                        jnp.tile(b2a[:_H], 4).reshape(1, 64),
                        jnp.tile(b2a[_H:], 4).reshape(1, 64),
                        jnp.tile(b2b[:_H], 4).reshape(1, 64),
                        jnp.tile(b2b[_H:], 4).reshape(1, 64))

    hel, heh, hol, hoh = _final_call(ier, ior, h2l.reshape(_P2, _H),
                                     h2h.reshape(_P2, _H))
    sel = jax.nn.one_hot((jnp.arange(128) % 64) // _H, 4, dtype=F32)
    out4 = _t7_call(hel.reshape(_F2 // 4, 64), hol.reshape(_F2 // 4, 64),
                    heh.reshape(_F2 // 4, 64), hoh.reshape(_F2 // 4, 64),
                    jnp.tile(Wp[:_H, 0], 4).reshape(1, 64),
                    jnp.tile(Wp[_H:, 0], 4).reshape(1, 64),
                    sel, bp.reshape(1, 1))
    return out4.reshape(_F2, 1)[:_F]
